# Initial kernel scaffold; baseline (speedup 1.0000x reference)
import jax
import jax.numpy as jnp
from jax.experimental import pallas as pl


def kernel(x, edge_attr, u, params, edge_index, batch):
    raise NotImplementedError("write your pallas kernel here")


# speedup vs baseline: 2.7803x; 2.7803x over previous
import functools

import jax
import jax.numpy as jnp
from jax import lax
from jax.experimental import pallas as pl
from jax.experimental.pallas import tpu as pltpu
from jax.experimental.pallas import tpu_sc as plsc

_E = 320000
_N = 10000
_D = 128
_NG = 64
_NW = 32
_EPW = _E // _NW
_GC = 80
_NCH = _EPW // _GC
_NP = 10240
_NPS = _NP // 16
_BE = 3200
_GRID = _E // _BE
_EPS = 1e-5
_SELU_A = 1.6732632423543772
_SELU_S = 1.0507009873554805

_INTERPRET = False


def _selu(t):
    return _SELU_S * jnp.where(t > 0, t, _SELU_A * (jnp.exp(t) - 1.0))


def _mesh():
    return plsc.VectorSubcoreMesh(core_axis_name="c", subcore_axis_name="s")


def _sc_gather(a, b, p, row3, col3):
    @functools.partial(
        pl.kernel,
        out_type=(jax.ShapeDtypeStruct((_E, _D), jnp.float32),
                  jax.ShapeDtypeStruct((_E, _D), jnp.float32)),
        mesh=_mesh(),
        scratch_types=[
            pltpu.VMEM((_NCH, _GC), jnp.int32),
            pltpu.VMEM((_NCH, _GC), jnp.int32),
            pltpu.VMEM((_GC, _D), jnp.float32),
            pltpu.VMEM((_GC, _D), jnp.float32),
        ],
    )
    def k(a_h, b_h, p_h, row_h, col_h, g_h, g2_h, rbuf, cbuf, bg, bp):
        wid = lax.axis_index("s") * 2 + lax.axis_index("c")
        base = wid * _EPW
        pltpu.sync_copy(row_h.at[wid], rbuf)
        pltpu.sync_copy(col_h.at[wid], cbuf)

        def body(i, carry):
            off = base + i * _GC
            pltpu.sync_copy(a_h.at[rbuf.at[i]], bg)
            pltpu.sync_copy(b_h.at[cbuf.at[i]], bg, add=True)
            pltpu.sync_copy(p_h.at[rbuf.at[i]], bp)
            pltpu.sync_copy(bg, g_h.at[pl.ds(off, _GC)])
            pltpu.sync_copy(bp, g2_h.at[pl.ds(off, _GC)])
            return carry

        lax.fori_loop(0, _NCH, body, 0)

    return k(a, b, p, row3, col3)


def _sc_scatter(h2, col3, znd):
    @functools.partial(
        pl.kernel,
        out_type=jax.ShapeDtypeStruct((2, _NP, _D), jnp.float32),
        mesh=_mesh(),
        scratch_types=[
            pltpu.VMEM((_NCH, _GC), jnp.int32),
            pltpu.VMEM((_GC, _D), jnp.float32),
            pltpu.VMEM_SHARED((_NP, _D), jnp.float32),
        ],
    )
    def k(h_h, col_h, z_h, out_h, cbuf, vbuf, shared):
        cid = lax.axis_index("c")
        sid = lax.axis_index("s")
        wid = sid * 2 + cid
        pltpu.sync_copy(col_h.at[wid], cbuf)
        pltpu.sync_copy(z_h.at[pl.ds(sid * _NPS, _NPS)],
                        shared.at[pl.ds(sid * _NPS, _NPS)])
        plsc.subcore_barrier()

        def body(i, carry):
            off = wid * _EPW + i * _GC
            pltpu.sync_copy(h_h.at[pl.ds(off, _GC)], vbuf)
            pltpu.sync_copy(vbuf, shared.at[cbuf.at[i]], add=True)
            return carry

        lax.fori_loop(0, _NCH, body, 0)
        plsc.subcore_barrier()
        pltpu.sync_copy(shared.at[pl.ds(sid * _NPS, _NPS)],
                        out_h.at[cid, pl.ds(sid * _NPS, _NPS)])

    return k(h2, col3, znd)


def _sc_counts(col3, z16, ones16):
    @functools.partial(
        pl.kernel,
        out_type=jax.ShapeDtypeStruct((2, _NP, _D), jnp.float32),
        mesh=_mesh(),
        scratch_types=[
            pltpu.VMEM((_NCH, _GC), jnp.int32),
            pltpu.VMEM((_GC, _D), jnp.float32),
            pltpu.VMEM_SHARED((_NP, _D), jnp.float32),
        ],
    )
    def k(col_h, z_h, ones_h, out_h, cbuf, obuf, shared):
        cid = lax.axis_index("c")
        sid = lax.axis_index("s")
        wid = sid * 2 + cid
        pltpu.sync_copy(col_h.at[wid], cbuf)
        pltpu.sync_copy(ones_h, obuf)
        pltpu.sync_copy(z_h.at[pl.ds(sid * _NPS, _NPS)],
                        shared.at[pl.ds(sid * _NPS, _NPS)])
        plsc.subcore_barrier()

        def body(i, carry):
            pltpu.sync_copy(obuf, shared.at[cbuf.at[i]], add=True)
            return carry

        lax.fori_loop(0, _NCH, body, 0)
        plsc.subcore_barrier()
        pltpu.sync_copy(shared.at[pl.ds(sid * _NPS, _NPS)],
                        out_h.at[cid, pl.ds(sid * _NPS, _NPS)])

    return k(col3, z16, ones16)


def _bspec_e(w):
    return pl.BlockSpec((_BE, w), lambda i: (i, 0))


def _bspec_c(shape):
    return pl.BlockSpec(shape, lambda i: (0,) * len(shape))


def _acc_stats(st_ref, z):
    st = jnp.concatenate([jnp.sum(z, axis=0, keepdims=True),
                          jnp.sum(z * z, axis=0, keepdims=True),
                          jnp.zeros((6, _D), jnp.float32)], axis=0)
    i = pl.program_id(0)

    @pl.when(i == 0)
    def _():
        st_ref[...] = st

    @pl.when(i != 0)
    def _():
        st_ref[...] = st_ref[...] + st


def _affine(st_ref, gm_ref, bt_ref):
    m = st_ref[0:1, :] * (1.0 / _E)
    v = st_ref[1:2, :] * (1.0 / _E) - m * m
    sc = gm_ref[...] * lax.rsqrt(v + _EPS)
    return sc, bt_ref[...] - m * sc


def _tc1(g, cin, wc, cv):
    de = cin.shape[1]

    def body(g_ref, c_ref, w_ref, v_ref, z_ref, st_ref):
        z = (g_ref[...]
             + jnp.dot(c_ref[...], w_ref[...],
                       preferred_element_type=jnp.float32)
             + v_ref[...])
        z_ref[...] = z
        _acc_stats(st_ref, z)

    return pl.pallas_call(
        body,
        grid=(_GRID,),
        in_specs=[_bspec_e(_D), _bspec_e(de), _bspec_c((de, _D)),
                  _bspec_c((1, _D))],
        out_specs=[_bspec_e(_D), _bspec_c((8, _D))],
        out_shape=[jax.ShapeDtypeStruct((_E, _D), jnp.float32),
                   jax.ShapeDtypeStruct((8, _D), jnp.float32)],
        interpret=_INTERPRET,
    )(g, cin, wc, cv)


def _tc2(z1, st1, gm, bt, w, b):
    def body(z_ref, st_ref, gm_ref, bt_ref, w_ref, b_ref, o_ref, so_ref):
        sc, sh = _affine(st_ref, gm_ref, bt_ref)
        e1 = _selu(z_ref[...] * sc + sh)
        z2 = jnp.dot(e1, w_ref[...],
                     preferred_element_type=jnp.float32) + b_ref[...]
        o_ref[...] = z2
        _acc_stats(so_ref, z2)

    return pl.pallas_call(
        body,
        grid=(_GRID,),
        in_specs=[_bspec_e(_D), _bspec_c((8, _D)), _bspec_c((1, _D)),
                  _bspec_c((1, _D)), _bspec_c((_D, _D)), _bspec_c((1, _D))],
        out_specs=[_bspec_e(_D), _bspec_c((8, _D))],
        out_shape=[jax.ShapeDtypeStruct((_E, _D), jnp.float32),
                   jax.ShapeDtypeStruct((8, _D), jnp.float32)],
        interpret=_INTERPRET,
    )(z1, st1, gm, bt, w, b)


def _tc3(z2, g2, st2, gm, bt, w3n, cv2, write_e2):
    def body_full(z_ref, g2_ref, st_ref, gm_ref, bt_ref, w_ref, v_ref,
                  e2_ref, h1_ref, so_ref):
        sc, sh = _affine(st_ref, gm_ref, bt_ref)
        e2 = _selu(z_ref[...] * sc + sh)
        e2_ref[...] = e2
        h1 = (g2_ref[...]
              + jnp.dot(e2, w_ref[...], preferred_element_type=jnp.float32)
              + v_ref[...])
        h1_ref[...] = h1
        _acc_stats(so_ref, h1)

    def body_nocarry(z_ref, g2_ref, st_ref, gm_ref, bt_ref, w_ref, v_ref,
                     h1_ref, so_ref):
        sc, sh = _affine(st_ref, gm_ref, bt_ref)
        e2 = _selu(z_ref[...] * sc + sh)
        h1 = (g2_ref[...]
              + jnp.dot(e2, w_ref[...], preferred_element_type=jnp.float32)
              + v_ref[...])
        h1_ref[...] = h1
        _acc_stats(so_ref, h1)

    in_specs = [_bspec_e(_D), _bspec_e(_D), _bspec_c((8, _D)),
                _bspec_c((1, _D)), _bspec_c((1, _D)), _bspec_c((_D, _D)),
                _bspec_c((1, _D))]
    if write_e2:
        return pl.pallas_call(
            body_full,
            grid=(_GRID,),
            in_specs=in_specs,
            out_specs=[_bspec_e(_D), _bspec_e(_D), _bspec_c((8, _D))],
            out_shape=[jax.ShapeDtypeStruct((_E, _D), jnp.float32),
                       jax.ShapeDtypeStruct((_E, _D), jnp.float32),
                       jax.ShapeDtypeStruct((8, _D), jnp.float32)],
            interpret=_INTERPRET,
        )(z2, g2, st2, gm, bt, w3n, cv2)
    h1, st3 = pl.pallas_call(
        body_nocarry,
        grid=(_GRID,),
        in_specs=in_specs,
        out_specs=[_bspec_e(_D), _bspec_c((8, _D))],
        out_shape=[jax.ShapeDtypeStruct((_E, _D), jnp.float32),
                   jax.ShapeDtypeStruct((8, _D), jnp.float32)],
        interpret=_INTERPRET,
    )(z2, g2, st2, gm, bt, w3n, cv2)
    return None, h1, st3


def _tc4(h1, st3, gm, bt, w, b):
    def body(h_ref, st_ref, gm_ref, bt_ref, w_ref, b_ref, o_ref, so_ref):
        sc, sh = _affine(st_ref, gm_ref, bt_ref)
        n1 = _selu(h_ref[...] * sc + sh)
        z4 = jnp.dot(n1, w_ref[...],
                     preferred_element_type=jnp.float32) + b_ref[...]
        o_ref[...] = z4
        _acc_stats(so_ref, z4)

    return pl.pallas_call(
        body,
        grid=(_GRID,),
        in_specs=[_bspec_e(_D), _bspec_c((8, _D)), _bspec_c((1, _D)),
                  _bspec_c((1, _D)), _bspec_c((_D, _D)), _bspec_c((1, _D))],
        out_specs=[_bspec_e(_D), _bspec_c((8, _D))],
        out_shape=[jax.ShapeDtypeStruct((_E, _D), jnp.float32),
                   jax.ShapeDtypeStruct((8, _D), jnp.float32)],
        interpret=_INTERPRET,
    )(h1, st3, gm, bt, w, b)


def _tc5(z4, st4, gm, bt):
    def body(z_ref, st_ref, gm_ref, bt_ref, o_ref):
        sc, sh = _affine(st_ref, gm_ref, bt_ref)
        o_ref[...] = _selu(z_ref[...] * sc + sh)

    return pl.pallas_call(
        body,
        grid=(_GRID,),
        in_specs=[_bspec_e(_D), _bspec_c((8, _D)), _bspec_c((1, _D)),
                  _bspec_c((1, _D))],
        out_specs=[_bspec_e(_D)],
        out_shape=[jax.ShapeDtypeStruct((_E, _D), jnp.float32)],
        interpret=_INTERPRET,
    )(z4, st4, gm, bt)[0]


def _onehot(b_ref):
    return (b_ref[...] == lax.broadcasted_iota(
        jnp.int32, (_N, _NG), 1)).astype(jnp.float32)


def _prep(x, u, b2d, wsrc, wdst, wu, b1v, wnx, bnv):
    def body(x_ref, u_ref, b_ref, ws, wd, wu_, b1_, wn_, bn_,
             a_ref, bb_ref, p_ref):
        xx = x_ref[...]
        oh = _onehot(b_ref)
        up = jnp.dot(u_ref[...], wu_[...], preferred_element_type=jnp.float32)
        a_ref[...] = (jnp.dot(xx, ws[...], preferred_element_type=jnp.float32)
                      + jnp.dot(oh, up, preferred_element_type=jnp.float32)
                      + b1_[...])
        bb_ref[...] = jnp.dot(xx, wd[...], preferred_element_type=jnp.float32)
        p_ref[...] = jnp.dot(xx, wn_[...],
                             preferred_element_type=jnp.float32) + bn_[...]

    return pl.pallas_call(
        body,
        out_shape=[jax.ShapeDtypeStruct((_N, _D), jnp.float32)] * 3,
        interpret=_INTERPRET,
    )(x, u, b2d, wsrc, wdst, wu, b1v, wnx, bnv)


def _bn_rows(t, gm, bt):
    m = jnp.mean(t, axis=0, keepdims=True)
    v = jnp.mean((t - m) ** 2, axis=0, keepdims=True)
    return (t - m) * lax.rsqrt(v + _EPS) * gm + bt


def _node(x, u, b2d, s2p, cp, wn3, bn3v, n2w, gw, nextw):
    has_next = nextw is not None

    def body(*refs):
        (x_ref, u_ref, b_ref, s_ref, c_ref, wn3_, bn3_,
         wax, wag, wau, ba1, ga1, bea1, wa2, ba2, ga2, bea2, wa3, ba3,
         wg1u, wg1n, bg1, gg1, beg1, wg2, bg2, gg2, beg2, wg3, bg3) = refs[:30]
        if has_next:
            nws, nwd, nwu, nb1, nwn, nbn = refs[30:36]
            un_ref, xn_ref, a_ref, bb_ref, p_ref = refs[36:]
        else:
            un_ref = refs[30]

        def dot(a, b):
            return jnp.dot(a, b, preferred_element_type=jnp.float32)

        s2 = s_ref[0, 0:_N, :] + s_ref[1, 0:_N, :]
        cc = c_ref[0, 0:_N, 0:1] + c_ref[1, 0:_N, 0:1]
        agg = (dot(s2, wn3_[...]) + cc * bn3_[...]) / jnp.maximum(cc, 1.0)
        oh = _onehot(b_ref)
        xx = x_ref[...]
        uu = u_ref[...]
        t = (dot(xx, wax[...]) + dot(agg, wag[...])
             + dot(oh, dot(uu, wau[...])) + ba1[...])
        t = _selu(_bn_rows(t, ga1[...], bea1[...]))
        t = dot(t, wa2[...]) + ba2[...]
        t = _selu(_bn_rows(t, ga2[...], bea2[...]))
        xn = dot(t, wa3[...]) + ba3[...]

        nc = lax.dot_general(oh, jnp.full((_N, 1), 1.0, jnp.float32),
                             (((0,), (0,)), ((), ())),
                             preferred_element_type=jnp.float32)
        ns = lax.dot_general(oh, xn, (((0,), (0,)), ((), ())),
                             preferred_element_type=jnp.float32)
        nmean = ns / jnp.maximum(nc, 1.0)
        g = dot(uu, wg1u[...]) + dot(nmean, wg1n[...]) + bg1[...]
        g = _selu(_bn_rows(g, gg1[...], beg1[...]))
        g = dot(g, wg2[...]) + bg2[...]
        g = _selu(_bn_rows(g, gg2[...], beg2[...]))
        un = dot(g, wg3[...]) + bg3[...]
        un_ref[...] = un
        if has_next:
            xn_ref[...] = xn
            a_ref[...] = (dot(xn, nws[...]) + dot(oh, dot(un, nwu[...]))
                          + nb1[...])
            bb_ref[...] = dot(xn, nwd[...])
            p_ref[...] = dot(xn, nwn[...]) + nbn[...]

    out_shape = [jax.ShapeDtypeStruct((_NG, _D), jnp.float32)]
    if has_next:
        out_shape += [jax.ShapeDtypeStruct((_N, _D), jnp.float32)] * 4
        args = (x, u, b2d, s2p, cp, wn3, bn3v) + n2w + gw + nextw
    else:
        args = (x, u, b2d, s2p, cp, wn3, bn3v) + n2w + gw
    res = pl.pallas_call(
        body, out_shape=out_shape,
        compiler_params=pltpu.CompilerParams(
            vmem_limit_bytes=100 * 1024 * 1024),
        interpret=_INTERPRET)(*args)
    return res if has_next else res[0]


def kernel(x, edge_attr, u, params, edge_index, batch):
    f32 = jnp.float32
    row3 = edge_index[0].reshape(_NW, _NCH, _GC)
    col3 = edge_index[1].reshape(_NW, _NCH, _GC)
    b2d = batch.reshape(_N, 1)
    znd = jnp.zeros((_NP, _D), f32)
    onesd = jnp.ones((_GC, _D), f32)
    cp = _sc_counts(col3, znd, onesd)

    def vec(v):
        return v.reshape(1, -1)

    xl, ul = x, u
    carry = None
    w3p = b3p = None
    nxt = None
    for l in range(3):
        lp = params[l]
        (w1, b1, g1, be1), (w2, b2, g2, be2), (w3, b3) = lp['edge']
        (wn1, bn1, gn1, ben1), (wn2, bn2, gn2, ben2), (wn3, bn3) = lp['node1']
        de = 16 if l == 0 else _D
        w1s, w1d = w1[0:_D], w1[_D:2 * _D]
        w1e, w1u = w1[2 * _D:2 * _D + de], w1[2 * _D + de:]
        if l == 0:
            a, bt, p = _prep(xl, ul, b2d, w1s, w1d, w1u, vec(b1),
                             wn1[:_D], vec(bn1))
            cin, wc, cv1 = edge_attr, w1e, jnp.zeros((1, _D), f32)
        else:
            a, bt, p = nxt
            cin = carry
            wc = w3p @ w1e
            cv1 = vec(b3p @ w1e)
        gth, gth2 = _sc_gather(a, bt, p, row3, col3)
        z1, st1 = _tc1(gth, cin, wc, cv1)
        z2, st2 = _tc2(z1, st1, vec(g1), vec(be1), w2, vec(b2))
        w3n = w3 @ wn1[_D:]
        cv2 = vec(b3 @ wn1[_D:])
        e2, h1, st3 = _tc3(z2, gth2, st2, vec(g2), vec(be2), w3n, cv2, l < 2)
        z4, st4 = _tc4(h1, st3, vec(gn1), vec(ben1), wn2, vec(bn2))
        h2 = _tc5(z4, st4, vec(gn2), vec(ben2))
        s2p = _sc_scatter(h2, col3, znd)
        (wa1, ba1, ga1, bea1), (wa2, ba2, ga2, bea2), (wa3, ba3) = lp['node2']
        (wg1, bg1, gg1, beg1), (wg2, bg2, gg2, beg2), (wg3, bg3) = lp['global']
        n2w = (wa1[:_D], wa1[_D:2 * _D], wa1[2 * _D:], vec(ba1), vec(ga1),
               vec(bea1), wa2, vec(ba2), vec(ga2), vec(bea2), wa3, vec(ba3))
        gw = (wg1[:_D], wg1[_D:], vec(bg1), vec(gg1), vec(beg1),
              wg2, vec(bg2), vec(gg2), vec(beg2), wg3, vec(bg3))
        if l < 2:
            nlp = params[l + 1]
            nw1, nb1 = nlp['edge'][0][0], nlp['edge'][0][1]
            nwn1, nbn1 = nlp['node1'][0][0], nlp['node1'][0][1]
            nextw = (nw1[0:_D], nw1[_D:2 * _D], nw1[3 * _D:], vec(nb1),
                     nwn1[:_D], vec(nbn1))
            un, xn, na, nb_, np_ = _node(xl, ul, b2d, s2p, cp, wn3, vec(bn3),
                                         n2w, gw, nextw)
            nxt = (na, nb_, np_)
            xl, ul = xn, un
            carry = e2
            w3p, b3p = w3, b3
        else:
            return _node(xl, ul, b2d, s2p, cp, wn3, vec(bn3), n2w, gw, None)


# speedup vs baseline: 3.3372x; 1.2003x over previous
import functools

import jax
import jax.numpy as jnp
from jax import lax
from jax.experimental import pallas as pl
from jax.experimental.pallas import tpu as pltpu
from jax.experimental.pallas import tpu_sc as plsc

_E = 320000
_N = 10000
_D = 128
_NG = 64
_NW = 32
_EPW = _E // _NW
_GC = 80
_NCH = _EPW // _GC
_SLOTS = 5
_SC_ = 40
_SNCH = _EPW // _SC_
_NP = 10240
_NPS = _NP // 16
_BE = 3200
_GRID = _E // _BE
_EPS = 1e-5
_SELU_A = 1.6732632423543772
_SELU_S = 1.0507009873554805

_INTERPRET = False


def _selu(t):
    return _SELU_S * jnp.where(t > 0, t, _SELU_A * (jnp.exp(t) - 1.0))


def _mesh():
    return plsc.VectorSubcoreMesh(core_axis_name="c", subcore_axis_name="s")


def _sc_gather(a, b, p, row4, col4):
    @functools.partial(
        pl.kernel,
        out_type=(jax.ShapeDtypeStruct((_E, _D), jnp.float32),
                  jax.ShapeDtypeStruct((_E, _D), jnp.float32)),
        mesh=_mesh(),
        scratch_types=[
            pltpu.VMEM((_SLOTS, _GC), jnp.int32),
            pltpu.VMEM((_SLOTS, _GC), jnp.int32),
            pltpu.VMEM((_SLOTS, _GC, _D), jnp.float32),
            pltpu.VMEM((_SLOTS, _GC, _D), jnp.float32),
            pltpu.SemaphoreType.DMA((_SLOTS,)),
            pltpu.SemaphoreType.DMA((_SLOTS,)),
        ],
    )
    def k(a_h, b_h, p_h, row_h, col_h, g_h, g2_h, ridx, cidx, bg, bp,
          sga, sgp):
        wid = lax.axis_index("s") * 2 + lax.axis_index("c")
        base = wid * _EPW

        def body(m, carry):
            pltpu.sync_copy(row_h.at[wid, m], ridx)
            pltpu.sync_copy(col_h.at[wid, m], cidx)
            da = [pltpu.async_copy(a_h.at[ridx.at[j]], bg.at[j],
                                   sga.at[j]) for j in range(_SLOTS)]
            dp = [pltpu.async_copy(p_h.at[ridx.at[j]], bp.at[j],
                                   sgp.at[j]) for j in range(_SLOTS)]
            db = []
            for j in range(_SLOTS):
                da[j].wait()
                db.append(pltpu.async_copy(b_h.at[cidx.at[j]],
                                           bg.at[j], sga.at[j], add=True))
            dw = []
            for j in range(_SLOTS):
                off = base + (m * _SLOTS + j) * _GC
                dp[j].wait()
                dw.append(pltpu.async_copy(bp.at[j],
                                           g2_h.at[pl.ds(off, _GC)],
                                           sgp.at[j]))
                db[j].wait()
                dw.append(pltpu.async_copy(bg.at[j],
                                           g_h.at[pl.ds(off, _GC)],
                                           sga.at[j]))
            for d in dw:
                d.wait()
            return carry

        lax.fori_loop(0, _NCH // _SLOTS, body, 0)

    return k(a, b, p, row4, col4)


def _sc_scatter(h2, col4s, znd):
    @functools.partial(
        pl.kernel,
        out_type=jax.ShapeDtypeStruct((2, _NP, _D), jnp.float32),
        mesh=_mesh(),
        scratch_types=[
            pltpu.VMEM((_SLOTS, _SC_), jnp.int32),
            pltpu.VMEM((_SLOTS, _SC_, _D), jnp.float32),
            pltpu.VMEM_SHARED((_NP, _D), jnp.float32),
            pltpu.SemaphoreType.DMA((_SLOTS,)),
            pltpu.SemaphoreType.DMA((_SLOTS,)),
        ],
    )
    def k(h_h, col_h, z_h, out_h, cidx, vbuf, shared, sld, sad):
        cid = lax.axis_index("c")
        sid = lax.axis_index("s")
        wid = sid * 2 + cid
        pltpu.sync_copy(z_h.at[pl.ds(sid * _NPS, _NPS)],
                        shared.at[pl.ds(sid * _NPS, _NPS)])
        plsc.subcore_barrier()

        def body(m, carry):
            pltpu.sync_copy(col_h.at[wid, m], cidx)
            dl = [pltpu.async_copy(
                h_h.at[pl.ds(wid * _EPW + (m * _SLOTS + j) * _SC_, _SC_)],
                vbuf.at[j], sld.at[j]) for j in range(_SLOTS)]
            da = []
            for j in range(_SLOTS):
                dl[j].wait()
                da.append(pltpu.async_copy(vbuf.at[j],
                                           shared.at[cidx.at[j]],
                                           sad.at[j], add=True))
            for d in da:
                d.wait()
            return carry

        lax.fori_loop(0, _SNCH // _SLOTS, body, 0)
        plsc.subcore_barrier()
        pltpu.sync_copy(shared.at[pl.ds(sid * _NPS, _NPS)],
                        out_h.at[cid, pl.ds(sid * _NPS, _NPS)])

    return k(h2, col4s, znd)


def _sc_counts(col4s, z16, ones16):
    @functools.partial(
        pl.kernel,
        out_type=jax.ShapeDtypeStruct((2, _NP, _D), jnp.float32),
        mesh=_mesh(),
        scratch_types=[
            pltpu.VMEM((_SLOTS, _SC_), jnp.int32),
            pltpu.VMEM((_SC_, _D), jnp.float32),
            pltpu.VMEM_SHARED((_NP, _D), jnp.float32),
            pltpu.SemaphoreType.DMA((_SLOTS,)),
        ],
    )
    def k(col_h, z_h, ones_h, out_h, cidx, obuf, shared, sad):
        cid = lax.axis_index("c")
        sid = lax.axis_index("s")
        wid = sid * 2 + cid
        pltpu.sync_copy(ones_h, obuf)
        pltpu.sync_copy(z_h.at[pl.ds(sid * _NPS, _NPS)],
                        shared.at[pl.ds(sid * _NPS, _NPS)])
        plsc.subcore_barrier()

        def body(m, carry):
            pltpu.sync_copy(col_h.at[wid, m], cidx)
            da = [pltpu.async_copy(obuf, shared.at[cidx.at[j]],
                                   sad.at[j], add=True)
                  for j in range(_SLOTS)]
            for d in da:
                d.wait()
            return carry

        lax.fori_loop(0, _SNCH // _SLOTS, body, 0)
        plsc.subcore_barrier()
        pltpu.sync_copy(shared.at[pl.ds(sid * _NPS, _NPS)],
                        out_h.at[cid, pl.ds(sid * _NPS, _NPS)])

    return k(col4s, z16, ones16)


def _bspec_e(w):
    return pl.BlockSpec((_BE, w), lambda i: (i, 0))


def _bspec_c(shape):
    return pl.BlockSpec(shape, lambda i: (0,) * len(shape))


def _acc_stats(st_ref, z):
    st = jnp.concatenate([jnp.sum(z, axis=0, keepdims=True),
                          jnp.sum(z * z, axis=0, keepdims=True),
                          jnp.zeros((6, _D), jnp.float32)], axis=0)
    i = pl.program_id(0)

    @pl.when(i == 0)
    def _():
        st_ref[...] = st

    @pl.when(i != 0)
    def _():
        st_ref[...] = st_ref[...] + st


def _affine(st_ref, gm_ref, bt_ref):
    m = st_ref[0:1, :] * (1.0 / _E)
    v = st_ref[1:2, :] * (1.0 / _E) - m * m
    sc = gm_ref[...] * lax.rsqrt(v + _EPS)
    return sc, bt_ref[...] - m * sc


def _tc1(g, cin, wc, cv):
    de = cin.shape[1]

    def body(g_ref, c_ref, w_ref, v_ref, z_ref, st_ref):
        z = (g_ref[...]
             + jnp.dot(c_ref[...], w_ref[...],
                       preferred_element_type=jnp.float32)
             + v_ref[...])
        z_ref[...] = z
        _acc_stats(st_ref, z)

    return pl.pallas_call(
        body,
        grid=(_GRID,),
        in_specs=[_bspec_e(_D), _bspec_e(de), _bspec_c((de, _D)),
                  _bspec_c((1, _D))],
        out_specs=[_bspec_e(_D), _bspec_c((8, _D))],
        out_shape=[jax.ShapeDtypeStruct((_E, _D), jnp.float32),
                   jax.ShapeDtypeStruct((8, _D), jnp.float32)],
        interpret=_INTERPRET,
    )(g, cin, wc, cv)


def _tc2(z1, st1, gm, bt, w, b):
    def body(z_ref, st_ref, gm_ref, bt_ref, w_ref, b_ref, o_ref, so_ref):
        sc, sh = _affine(st_ref, gm_ref, bt_ref)
        e1 = _selu(z_ref[...] * sc + sh)
        z2 = jnp.dot(e1, w_ref[...],
                     preferred_element_type=jnp.float32) + b_ref[...]
        o_ref[...] = z2
        _acc_stats(so_ref, z2)

    return pl.pallas_call(
        body,
        grid=(_GRID,),
        in_specs=[_bspec_e(_D), _bspec_c((8, _D)), _bspec_c((1, _D)),
                  _bspec_c((1, _D)), _bspec_c((_D, _D)), _bspec_c((1, _D))],
        out_specs=[_bspec_e(_D), _bspec_c((8, _D))],
        out_shape=[jax.ShapeDtypeStruct((_E, _D), jnp.float32),
                   jax.ShapeDtypeStruct((8, _D), jnp.float32)],
        interpret=_INTERPRET,
    )(z1, st1, gm, bt, w, b)


def _tc3(z2, g2, st2, gm, bt, w3n, cv2, write_e2):
    def body_full(z_ref, g2_ref, st_ref, gm_ref, bt_ref, w_ref, v_ref,
                  e2_ref, h1_ref, so_ref):
        sc, sh = _affine(st_ref, gm_ref, bt_ref)
        e2 = _selu(z_ref[...] * sc + sh)
        e2_ref[...] = e2
        h1 = (g2_ref[...]
              + jnp.dot(e2, w_ref[...], preferred_element_type=jnp.float32)
              + v_ref[...])
        h1_ref[...] = h1
        _acc_stats(so_ref, h1)

    def body_nocarry(z_ref, g2_ref, st_ref, gm_ref, bt_ref, w_ref, v_ref,
                     h1_ref, so_ref):
        sc, sh = _affine(st_ref, gm_ref, bt_ref)
        e2 = _selu(z_ref[...] * sc + sh)
        h1 = (g2_ref[...]
              + jnp.dot(e2, w_ref[...], preferred_element_type=jnp.float32)
              + v_ref[...])
        h1_ref[...] = h1
        _acc_stats(so_ref, h1)

    in_specs = [_bspec_e(_D), _bspec_e(_D), _bspec_c((8, _D)),
                _bspec_c((1, _D)), _bspec_c((1, _D)), _bspec_c((_D, _D)),
                _bspec_c((1, _D))]
    if write_e2:
        return pl.pallas_call(
            body_full,
            grid=(_GRID,),
            in_specs=in_specs,
            out_specs=[_bspec_e(_D), _bspec_e(_D), _bspec_c((8, _D))],
            out_shape=[jax.ShapeDtypeStruct((_E, _D), jnp.float32),
                       jax.ShapeDtypeStruct((_E, _D), jnp.float32),
                       jax.ShapeDtypeStruct((8, _D), jnp.float32)],
            interpret=_INTERPRET,
        )(z2, g2, st2, gm, bt, w3n, cv2)
    h1, st3 = pl.pallas_call(
        body_nocarry,
        grid=(_GRID,),
        in_specs=in_specs,
        out_specs=[_bspec_e(_D), _bspec_c((8, _D))],
        out_shape=[jax.ShapeDtypeStruct((_E, _D), jnp.float32),
                   jax.ShapeDtypeStruct((8, _D), jnp.float32)],
        interpret=_INTERPRET,
    )(z2, g2, st2, gm, bt, w3n, cv2)
    return None, h1, st3


def _tc4(h1, st3, gm, bt, w, b):
    def body(h_ref, st_ref, gm_ref, bt_ref, w_ref, b_ref, o_ref, so_ref):
        sc, sh = _affine(st_ref, gm_ref, bt_ref)
        n1 = _selu(h_ref[...] * sc + sh)
        z4 = jnp.dot(n1, w_ref[...],
                     preferred_element_type=jnp.float32) + b_ref[...]
        o_ref[...] = z4
        _acc_stats(so_ref, z4)

    return pl.pallas_call(
        body,
        grid=(_GRID,),
        in_specs=[_bspec_e(_D), _bspec_c((8, _D)), _bspec_c((1, _D)),
                  _bspec_c((1, _D)), _bspec_c((_D, _D)), _bspec_c((1, _D))],
        out_specs=[_bspec_e(_D), _bspec_c((8, _D))],
        out_shape=[jax.ShapeDtypeStruct((_E, _D), jnp.float32),
                   jax.ShapeDtypeStruct((8, _D), jnp.float32)],
        interpret=_INTERPRET,
    )(h1, st3, gm, bt, w, b)


def _tc5(z4, st4, gm, bt):
    def body(z_ref, st_ref, gm_ref, bt_ref, o_ref):
        sc, sh = _affine(st_ref, gm_ref, bt_ref)
        o_ref[...] = _selu(z_ref[...] * sc + sh)

    return pl.pallas_call(
        body,
        grid=(_GRID,),
        in_specs=[_bspec_e(_D), _bspec_c((8, _D)), _bspec_c((1, _D)),
                  _bspec_c((1, _D))],
        out_specs=[_bspec_e(_D)],
        out_shape=[jax.ShapeDtypeStruct((_E, _D), jnp.float32)],
        interpret=_INTERPRET,
    )(z4, st4, gm, bt)[0]


def _onehot(b_ref):
    return (b_ref[...] == lax.broadcasted_iota(
        jnp.int32, (_N, _NG), 1)).astype(jnp.float32)


def _prep(x, u, b2d, wsrc, wdst, wu, b1v, wnx, bnv):
    def body(x_ref, u_ref, b_ref, ws, wd, wu_, b1_, wn_, bn_,
             a_ref, bb_ref, p_ref):
        xx = x_ref[...]
        oh = _onehot(b_ref)
        up = jnp.dot(u_ref[...], wu_[...], preferred_element_type=jnp.float32)
        a_ref[...] = (jnp.dot(xx, ws[...], preferred_element_type=jnp.float32)
                      + jnp.dot(oh, up, preferred_element_type=jnp.float32)
                      + b1_[...])
        bb_ref[...] = jnp.dot(xx, wd[...], preferred_element_type=jnp.float32)
        p_ref[...] = jnp.dot(xx, wn_[...],
                             preferred_element_type=jnp.float32) + bn_[...]

    return pl.pallas_call(
        body,
        out_shape=[jax.ShapeDtypeStruct((_N, _D), jnp.float32)] * 3,
        interpret=_INTERPRET,
    )(x, u, b2d, wsrc, wdst, wu, b1v, wnx, bnv)


def _bn_rows(t, gm, bt):
    m = jnp.mean(t, axis=0, keepdims=True)
    v = jnp.mean((t - m) ** 2, axis=0, keepdims=True)
    return (t - m) * lax.rsqrt(v + _EPS) * gm + bt


def _node(x, u, b2d, s2p, cp, wn3, bn3v, n2w, gw, nextw):
    has_next = nextw is not None

    def body(*refs):
        (x_ref, u_ref, b_ref, s_ref, c_ref, wn3_, bn3_,
         wax, wag, wau, ba1, ga1, bea1, wa2, ba2, ga2, bea2, wa3, ba3,
         wg1u, wg1n, bg1, gg1, beg1, wg2, bg2, gg2, beg2, wg3, bg3) = refs[:30]
        if has_next:
            nws, nwd, nwu, nb1, nwn, nbn = refs[30:36]
            un_ref, xn_ref, a_ref, bb_ref, p_ref = refs[36:]
        else:
            un_ref = refs[30]

        def dot(a, b):
            return jnp.dot(a, b, preferred_element_type=jnp.float32)

        s2 = s_ref[0, 0:_N, :] + s_ref[1, 0:_N, :]
        cc = c_ref[0, 0:_N, 0:1] + c_ref[1, 0:_N, 0:1]
        agg = (dot(s2, wn3_[...]) + cc * bn3_[...]) / jnp.maximum(cc, 1.0)
        oh = _onehot(b_ref)
        xx = x_ref[...]
        uu = u_ref[...]
        t = (dot(xx, wax[...]) + dot(agg, wag[...])
             + dot(oh, dot(uu, wau[...])) + ba1[...])
        t = _selu(_bn_rows(t, ga1[...], bea1[...]))
        t = dot(t, wa2[...]) + ba2[...]
        t = _selu(_bn_rows(t, ga2[...], bea2[...]))
        xn = dot(t, wa3[...]) + ba3[...]

        nc = lax.dot_general(oh, jnp.full((_N, 1), 1.0, jnp.float32),
                             (((0,), (0,)), ((), ())),
                             preferred_element_type=jnp.float32)
        ns = lax.dot_general(oh, xn, (((0,), (0,)), ((), ())),
                             preferred_element_type=jnp.float32)
        nmean = ns / jnp.maximum(nc, 1.0)
        g = dot(uu, wg1u[...]) + dot(nmean, wg1n[...]) + bg1[...]
        g = _selu(_bn_rows(g, gg1[...], beg1[...]))
        g = dot(g, wg2[...]) + bg2[...]
        g = _selu(_bn_rows(g, gg2[...], beg2[...]))
        un = dot(g, wg3[...]) + bg3[...]
        un_ref[...] = un
        if has_next:
            xn_ref[...] = xn
            a_ref[...] = (dot(xn, nws[...]) + dot(oh, dot(un, nwu[...]))
                          + nb1[...])
            bb_ref[...] = dot(xn, nwd[...])
            p_ref[...] = dot(xn, nwn[...]) + nbn[...]

    out_shape = [jax.ShapeDtypeStruct((_NG, _D), jnp.float32)]
    if has_next:
        out_shape += [jax.ShapeDtypeStruct((_N, _D), jnp.float32)] * 4
        args = (x, u, b2d, s2p, cp, wn3, bn3v) + n2w + gw + nextw
    else:
        args = (x, u, b2d, s2p, cp, wn3, bn3v) + n2w + gw
    res = pl.pallas_call(
        body, out_shape=out_shape,
        compiler_params=pltpu.CompilerParams(
            vmem_limit_bytes=100 * 1024 * 1024),
        interpret=_INTERPRET)(*args)
    return res if has_next else res[0]


def kernel(x, edge_attr, u, params, edge_index, batch):
    f32 = jnp.float32
    row4 = edge_index[0].reshape(_NW, _NCH // _SLOTS, _SLOTS, _GC)
    col4 = edge_index[1].reshape(_NW, _NCH // _SLOTS, _SLOTS, _GC)
    col4s = edge_index[1].reshape(_NW, _SNCH // _SLOTS, _SLOTS, _SC_)
    b2d = batch.reshape(_N, 1)
    znd = jnp.zeros((_NP, _D), f32)
    onesd = jnp.ones((_SC_, _D), f32)
    cp = _sc_counts(col4s, znd, onesd)

    def vec(v):
        return v.reshape(1, -1)

    xl, ul = x, u
    carry = None
    w3p = b3p = None
    nxt = None
    for l in range(3):
        lp = params[l]
        (w1, b1, g1, be1), (w2, b2, g2, be2), (w3, b3) = lp['edge']
        (wn1, bn1, gn1, ben1), (wn2, bn2, gn2, ben2), (wn3, bn3) = lp['node1']
        de = 16 if l == 0 else _D
        w1s, w1d = w1[0:_D], w1[_D:2 * _D]
        w1e, w1u = w1[2 * _D:2 * _D + de], w1[2 * _D + de:]
        if l == 0:
            a, bt, p = _prep(xl, ul, b2d, w1s, w1d, w1u, vec(b1),
                             wn1[:_D], vec(bn1))
            cin, wc, cv1 = edge_attr, w1e, jnp.zeros((1, _D), f32)
        else:
            a, bt, p = nxt
            cin = carry
            wc = w3p @ w1e
            cv1 = vec(b3p @ w1e)
        gth, gth2 = _sc_gather(a, bt, p, row4, col4)
        z1, st1 = _tc1(gth, cin, wc, cv1)
        z2, st2 = _tc2(z1, st1, vec(g1), vec(be1), w2, vec(b2))
        w3n = w3 @ wn1[_D:]
        cv2 = vec(b3 @ wn1[_D:])
        e2, h1, st3 = _tc3(z2, gth2, st2, vec(g2), vec(be2), w3n, cv2, l < 2)
        z4, st4 = _tc4(h1, st3, vec(gn1), vec(ben1), wn2, vec(bn2))
        h2 = _tc5(z4, st4, vec(gn2), vec(ben2))
        s2p = _sc_scatter(h2, col4s, znd)
        (wa1, ba1, ga1, bea1), (wa2, ba2, ga2, bea2), (wa3, ba3) = lp['node2']
        (wg1, bg1, gg1, beg1), (wg2, bg2, gg2, beg2), (wg3, bg3) = lp['global']
        n2w = (wa1[:_D], wa1[_D:2 * _D], wa1[2 * _D:], vec(ba1), vec(ga1),
               vec(bea1), wa2, vec(ba2), vec(ga2), vec(bea2), wa3, vec(ba3))
        gw = (wg1[:_D], wg1[_D:], vec(bg1), vec(gg1), vec(beg1),
              wg2, vec(bg2), vec(gg2), vec(beg2), wg3, vec(bg3))
        if l < 2:
            nlp = params[l + 1]
            nw1, nb1 = nlp['edge'][0][0], nlp['edge'][0][1]
            nwn1, nbn1 = nlp['node1'][0][0], nlp['node1'][0][1]
            nextw = (nw1[0:_D], nw1[_D:2 * _D], nw1[3 * _D:], vec(nb1),
                     nwn1[:_D], vec(nbn1))
            un, xn, na, nb_, np_ = _node(xl, ul, b2d, s2p, cp, wn3, vec(bn3),
                                         n2w, gw, nextw)
            nxt = (na, nb_, np_)
            xl, ul = xn, un
            carry = e2
            w3p, b3p = w3, b3
        else:
            return _node(xl, ul, b2d, s2p, cp, wn3, vec(bn3), n2w, gw, None)


# speedup vs baseline: 3.6964x; 1.1076x over previous
import functools

import jax
import jax.numpy as jnp
from jax import lax
from jax.experimental import pallas as pl
from jax.experimental.pallas import tpu as pltpu
from jax.experimental.pallas import tpu_sc as plsc

_E = 320000
_N = 10000
_D = 128
_NG = 64
_NW = 32
_EPW = _E // _NW
_GC = 80
_NCH = _EPW // _GC
_SLOTS = 5
_SC_ = 40
_SNCH = _EPW // _SC_
_NP = 10240
_NPS = _NP // 16
_BE = 3200
_GRID = _E // _BE
_EPS = 1e-5
_SELU_A = 1.6732632423543772
_SELU_S = 1.0507009873554805

_INTERPRET = False


def _selu(t):
    return _SELU_S * jnp.where(t > 0, t, _SELU_A * (jnp.exp(t) - 1.0))


def _mesh():
    return plsc.VectorSubcoreMesh(core_axis_name="c", subcore_axis_name="s")


def _sc_gather(a, b, p, row4, col4):
    @functools.partial(
        pl.kernel,
        out_type=(jax.ShapeDtypeStruct((_E, _D), jnp.float32),
                  jax.ShapeDtypeStruct((_E, _D), jnp.float32)),
        mesh=_mesh(),
        scratch_types=[
            pltpu.VMEM((_SLOTS, _GC), jnp.int32),
            pltpu.VMEM((_SLOTS, _GC), jnp.int32),
            pltpu.VMEM((_SLOTS, _GC, _D), jnp.float32),
            pltpu.VMEM((_SLOTS, _GC, _D), jnp.float32),
            pltpu.SemaphoreType.DMA((_SLOTS,)),
            pltpu.SemaphoreType.DMA((_SLOTS,)),
        ],
    )
    def k(a_h, b_h, p_h, row_h, col_h, g_h, g2_h, ridx, cidx, bg, bp,
          sga, sgp):
        wid = lax.axis_index("s") * 2 + lax.axis_index("c")
        base = wid * _EPW

        def body(m, carry):
            pltpu.sync_copy(row_h.at[wid, m], ridx)
            pltpu.sync_copy(col_h.at[wid, m], cidx)
            da = [pltpu.async_copy(a_h.at[ridx.at[j]], bg.at[j],
                                   sga.at[j]) for j in range(_SLOTS)]
            dp = [pltpu.async_copy(p_h.at[ridx.at[j]], bp.at[j],
                                   sgp.at[j]) for j in range(_SLOTS)]
            db = []
            for j in range(_SLOTS):
                da[j].wait()
                db.append(pltpu.async_copy(b_h.at[cidx.at[j]],
                                           bg.at[j], sga.at[j], add=True))
            dw = []
            for j in range(_SLOTS):
                off = base + (m * _SLOTS + j) * _GC
                dp[j].wait()
                dw.append(pltpu.async_copy(bp.at[j],
                                           g2_h.at[pl.ds(off, _GC)],
                                           sgp.at[j]))
                db[j].wait()
                dw.append(pltpu.async_copy(bg.at[j],
                                           g_h.at[pl.ds(off, _GC)],
                                           sga.at[j]))
            for d in dw:
                d.wait()
            return carry

        lax.fori_loop(0, _NCH // _SLOTS, body, 0)

    return k(a, b, p, row4, col4)


def _sc_scatter(h2, col4s, znd):
    @functools.partial(
        pl.kernel,
        out_type=jax.ShapeDtypeStruct((2, _NP, _D), jnp.float32),
        mesh=_mesh(),
        scratch_types=[
            pltpu.VMEM((_SLOTS, _SC_), jnp.int32),
            pltpu.VMEM((_SLOTS, _SC_, _D), jnp.float32),
            pltpu.VMEM_SHARED((_NP, _D), jnp.float32),
            pltpu.SemaphoreType.DMA((_SLOTS,)),
            pltpu.SemaphoreType.DMA((_SLOTS,)),
        ],
    )
    def k(h_h, col_h, z_h, out_h, cidx, vbuf, shared, sld, sad):
        cid = lax.axis_index("c")
        sid = lax.axis_index("s")
        wid = sid * 2 + cid
        pltpu.sync_copy(z_h.at[pl.ds(sid * _NPS, _NPS)],
                        shared.at[pl.ds(sid * _NPS, _NPS)])
        plsc.subcore_barrier()

        def body(m, carry):
            pltpu.sync_copy(col_h.at[wid, m], cidx)
            dl = [pltpu.async_copy(
                h_h.at[pl.ds(wid * _EPW + (m * _SLOTS + j) * _SC_, _SC_)],
                vbuf.at[j], sld.at[j]) for j in range(_SLOTS)]
            da = []
            for j in range(_SLOTS):
                dl[j].wait()
                da.append(pltpu.async_copy(vbuf.at[j],
                                           shared.at[cidx.at[j]],
                                           sad.at[j], add=True))
            for d in da:
                d.wait()
            return carry

        lax.fori_loop(0, _SNCH // _SLOTS, body, 0)
        plsc.subcore_barrier()
        pltpu.sync_copy(shared.at[pl.ds(sid * _NPS, _NPS)],
                        out_h.at[cid, pl.ds(sid * _NPS, _NPS)])

    return k(h2, col4s, znd)


def _sc_counts(col4s, z16, ones16):
    @functools.partial(
        pl.kernel,
        out_type=jax.ShapeDtypeStruct((2, _NP, _D), jnp.float32),
        mesh=_mesh(),
        scratch_types=[
            pltpu.VMEM((_SLOTS, _SC_), jnp.int32),
            pltpu.VMEM((_SC_, _D), jnp.float32),
            pltpu.VMEM_SHARED((_NP, _D), jnp.float32),
            pltpu.SemaphoreType.DMA((_SLOTS,)),
        ],
    )
    def k(col_h, z_h, ones_h, out_h, cidx, obuf, shared, sad):
        cid = lax.axis_index("c")
        sid = lax.axis_index("s")
        wid = sid * 2 + cid
        pltpu.sync_copy(ones_h, obuf)
        pltpu.sync_copy(z_h.at[pl.ds(sid * _NPS, _NPS)],
                        shared.at[pl.ds(sid * _NPS, _NPS)])
        plsc.subcore_barrier()

        def body(m, carry):
            pltpu.sync_copy(col_h.at[wid, m], cidx)
            da = [pltpu.async_copy(obuf, shared.at[cidx.at[j]],
                                   sad.at[j], add=True)
                  for j in range(_SLOTS)]
            for d in da:
                d.wait()
            return carry

        lax.fori_loop(0, _SNCH // _SLOTS, body, 0)
        plsc.subcore_barrier()
        pltpu.sync_copy(shared.at[pl.ds(sid * _NPS, _NPS)],
                        out_h.at[cid, pl.ds(sid * _NPS, _NPS)])

    return k(col4s, z16, ones16)


def _bspec_e(w):
    return pl.BlockSpec((_BE, w), lambda i: (i, 0))


def _bspec_c(shape):
    return pl.BlockSpec(shape, lambda i: (0,) * len(shape))


def _f32(ref):
    return ref[...].astype(jnp.float32)


def _acc_stats(st_ref, z):
    st = jnp.concatenate([jnp.sum(z, axis=0, keepdims=True),
                          jnp.sum(z * z, axis=0, keepdims=True),
                          jnp.zeros((6, _D), jnp.float32)], axis=0)
    i = pl.program_id(0)

    @pl.when(i == 0)
    def _():
        st_ref[...] = st

    @pl.when(i != 0)
    def _():
        st_ref[...] = st_ref[...] + st


def _affine(st_ref, gm_ref, bt_ref):
    m = st_ref[0:1, :] * (1.0 / _E)
    v = st_ref[1:2, :] * (1.0 / _E) - m * m
    sc = gm_ref[...] * lax.rsqrt(v + _EPS)
    return sc, bt_ref[...] - m * sc


def _tc1(g, cin, wc, cv):
    de = cin.shape[1]

    def body(g_ref, c_ref, w_ref, v_ref, z_ref, st_ref):
        z = (g_ref[...]
             + jnp.dot(_f32(c_ref), w_ref[...],
                       preferred_element_type=jnp.float32)
             + v_ref[...])
        z_ref[...] = z.astype(jnp.bfloat16)
        _acc_stats(st_ref, z)

    return pl.pallas_call(
        body,
        grid=(_GRID,),
        in_specs=[_bspec_e(_D), _bspec_e(de), _bspec_c((de, _D)),
                  _bspec_c((1, _D))],
        out_specs=[_bspec_e(_D), _bspec_c((8, _D))],
        out_shape=[jax.ShapeDtypeStruct((_E, _D), jnp.bfloat16),
                   jax.ShapeDtypeStruct((8, _D), jnp.float32)],
        interpret=_INTERPRET,
    )(g, cin, wc, cv)


def _tc2(z1, st1, gm, bt, w, b):
    def body(z_ref, st_ref, gm_ref, bt_ref, w_ref, b_ref, o_ref, so_ref):
        sc, sh = _affine(st_ref, gm_ref, bt_ref)
        e1 = _selu(_f32(z_ref) * sc + sh)
        z2 = jnp.dot(e1, w_ref[...],
                     preferred_element_type=jnp.float32) + b_ref[...]
        o_ref[...] = z2.astype(jnp.bfloat16)
        _acc_stats(so_ref, z2)

    return pl.pallas_call(
        body,
        grid=(_GRID,),
        in_specs=[_bspec_e(_D), _bspec_c((8, _D)), _bspec_c((1, _D)),
                  _bspec_c((1, _D)), _bspec_c((_D, _D)), _bspec_c((1, _D))],
        out_specs=[_bspec_e(_D), _bspec_c((8, _D))],
        out_shape=[jax.ShapeDtypeStruct((_E, _D), jnp.bfloat16),
                   jax.ShapeDtypeStruct((8, _D), jnp.float32)],
        interpret=_INTERPRET,
    )(z1, st1, gm, bt, w, b)


def _tc3(z2, g2, st2, gm, bt, w3n, cv2, write_e2):
    def body_full(z_ref, g2_ref, st_ref, gm_ref, bt_ref, w_ref, v_ref,
                  e2_ref, h1_ref, so_ref):
        sc, sh = _affine(st_ref, gm_ref, bt_ref)
        e2 = _selu(_f32(z_ref) * sc + sh)
        e2_ref[...] = e2.astype(jnp.bfloat16)
        h1 = (g2_ref[...]
              + jnp.dot(e2, w_ref[...], preferred_element_type=jnp.float32)
              + v_ref[...])
        h1_ref[...] = h1.astype(jnp.bfloat16)
        _acc_stats(so_ref, h1)

    def body_nocarry(z_ref, g2_ref, st_ref, gm_ref, bt_ref, w_ref, v_ref,
                     h1_ref, so_ref):
        sc, sh = _affine(st_ref, gm_ref, bt_ref)
        e2 = _selu(_f32(z_ref) * sc + sh)
        h1 = (g2_ref[...]
              + jnp.dot(e2, w_ref[...], preferred_element_type=jnp.float32)
              + v_ref[...])
        h1_ref[...] = h1.astype(jnp.bfloat16)
        _acc_stats(so_ref, h1)

    in_specs = [_bspec_e(_D), _bspec_e(_D), _bspec_c((8, _D)),
                _bspec_c((1, _D)), _bspec_c((1, _D)), _bspec_c((_D, _D)),
                _bspec_c((1, _D))]
    if write_e2:
        return pl.pallas_call(
            body_full,
            grid=(_GRID,),
            in_specs=in_specs,
            out_specs=[_bspec_e(_D), _bspec_e(_D), _bspec_c((8, _D))],
            out_shape=[jax.ShapeDtypeStruct((_E, _D), jnp.bfloat16),
                       jax.ShapeDtypeStruct((_E, _D), jnp.bfloat16),
                       jax.ShapeDtypeStruct((8, _D), jnp.float32)],
            interpret=_INTERPRET,
        )(z2, g2, st2, gm, bt, w3n, cv2)
    h1, st3 = pl.pallas_call(
        body_nocarry,
        grid=(_GRID,),
        in_specs=in_specs,
        out_specs=[_bspec_e(_D), _bspec_c((8, _D))],
        out_shape=[jax.ShapeDtypeStruct((_E, _D), jnp.bfloat16),
                   jax.ShapeDtypeStruct((8, _D), jnp.float32)],
        interpret=_INTERPRET,
    )(z2, g2, st2, gm, bt, w3n, cv2)
    return None, h1, st3


def _tc4(h1, st3, gm, bt, w, b):
    def body(h_ref, st_ref, gm_ref, bt_ref, w_ref, b_ref, o_ref, so_ref):
        sc, sh = _affine(st_ref, gm_ref, bt_ref)
        n1 = _selu(_f32(h_ref) * sc + sh)
        z4 = jnp.dot(n1, w_ref[...],
                     preferred_element_type=jnp.float32) + b_ref[...]
        o_ref[...] = z4.astype(jnp.bfloat16)
        _acc_stats(so_ref, z4)

    return pl.pallas_call(
        body,
        grid=(_GRID,),
        in_specs=[_bspec_e(_D), _bspec_c((8, _D)), _bspec_c((1, _D)),
                  _bspec_c((1, _D)), _bspec_c((_D, _D)), _bspec_c((1, _D))],
        out_specs=[_bspec_e(_D), _bspec_c((8, _D))],
        out_shape=[jax.ShapeDtypeStruct((_E, _D), jnp.bfloat16),
                   jax.ShapeDtypeStruct((8, _D), jnp.float32)],
        interpret=_INTERPRET,
    )(h1, st3, gm, bt, w, b)


def _tc5(z4, st4, gm, bt):
    def body(z_ref, st_ref, gm_ref, bt_ref, o_ref):
        sc, sh = _affine(st_ref, gm_ref, bt_ref)
        o_ref[...] = _selu(_f32(z_ref) * sc + sh)

    return pl.pallas_call(
        body,
        grid=(_GRID,),
        in_specs=[_bspec_e(_D), _bspec_c((8, _D)), _bspec_c((1, _D)),
                  _bspec_c((1, _D))],
        out_specs=[_bspec_e(_D)],
        out_shape=[jax.ShapeDtypeStruct((_E, _D), jnp.float32)],
        interpret=_INTERPRET,
    )(z4, st4, gm, bt)[0]


def _onehot(b_ref):
    return (b_ref[...] == lax.broadcasted_iota(
        jnp.int32, (_N, _NG), 1)).astype(jnp.float32)


def _prep(x, u, b2d, wsrc, wdst, wu, b1v, wnx, bnv):
    def body(x_ref, u_ref, b_ref, ws, wd, wu_, b1_, wn_, bn_,
             a_ref, bb_ref, p_ref):
        xx = x_ref[...]
        oh = _onehot(b_ref)
        up = jnp.dot(u_ref[...], wu_[...], preferred_element_type=jnp.float32)
        a_ref[...] = (jnp.dot(xx, ws[...], preferred_element_type=jnp.float32)
                      + jnp.dot(oh, up, preferred_element_type=jnp.float32)
                      + b1_[...])
        bb_ref[...] = jnp.dot(xx, wd[...], preferred_element_type=jnp.float32)
        p_ref[...] = jnp.dot(xx, wn_[...],
                             preferred_element_type=jnp.float32) + bn_[...]

    return pl.pallas_call(
        body,
        out_shape=[jax.ShapeDtypeStruct((_N, _D), jnp.float32)] * 3,
        interpret=_INTERPRET,
    )(x, u, b2d, wsrc, wdst, wu, b1v, wnx, bnv)


def _bn_rows(t, gm, bt):
    m = jnp.mean(t, axis=0, keepdims=True)
    v = jnp.mean((t - m) ** 2, axis=0, keepdims=True)
    return (t - m) * lax.rsqrt(v + _EPS) * gm + bt


def _node(x, u, b2d, s2p, cp, wn3, bn3v, n2w, gw, nextw):
    has_next = nextw is not None

    def body(*refs):
        (x_ref, u_ref, b_ref, s_ref, c_ref, wn3_, bn3_,
         wax, wag, wau, ba1, ga1, bea1, wa2, ba2, ga2, bea2, wa3, ba3,
         wg1u, wg1n, bg1, gg1, beg1, wg2, bg2, gg2, beg2, wg3, bg3) = refs[:30]
        if has_next:
            nws, nwd, nwu, nb1, nwn, nbn = refs[30:36]
            un_ref, xn_ref, a_ref, bb_ref, p_ref = refs[36:]
        else:
            un_ref = refs[30]

        def dot(a, b):
            return jnp.dot(a, b, preferred_element_type=jnp.float32)

        s2 = s_ref[0, 0:_N, :] + s_ref[1, 0:_N, :]
        cc = c_ref[0, 0:_N, 0:1] + c_ref[1, 0:_N, 0:1]
        agg = (dot(s2, wn3_[...]) + cc * bn3_[...]) / jnp.maximum(cc, 1.0)
        oh = _onehot(b_ref)
        xx = x_ref[...]
        uu = u_ref[...]
        t = (dot(xx, wax[...]) + dot(agg, wag[...])
             + dot(oh, dot(uu, wau[...])) + ba1[...])
        t = _selu(_bn_rows(t, ga1[...], bea1[...]))
        t = dot(t, wa2[...]) + ba2[...]
        t = _selu(_bn_rows(t, ga2[...], bea2[...]))
        xn = dot(t, wa3[...]) + ba3[...]

        nc = lax.dot_general(oh, jnp.full((_N, 1), 1.0, jnp.float32),
                             (((0,), (0,)), ((), ())),
                             preferred_element_type=jnp.float32)
        ns = lax.dot_general(oh, xn, (((0,), (0,)), ((), ())),
                             preferred_element_type=jnp.float32)
        nmean = ns / jnp.maximum(nc, 1.0)
        g = dot(uu, wg1u[...]) + dot(nmean, wg1n[...]) + bg1[...]
        g = _selu(_bn_rows(g, gg1[...], beg1[...]))
        g = dot(g, wg2[...]) + bg2[...]
        g = _selu(_bn_rows(g, gg2[...], beg2[...]))
        un = dot(g, wg3[...]) + bg3[...]
        un_ref[...] = un
        if has_next:
            xn_ref[...] = xn
            a_ref[...] = (dot(xn, nws[...]) + dot(oh, dot(un, nwu[...]))
                          + nb1[...])
            bb_ref[...] = dot(xn, nwd[...])
            p_ref[...] = dot(xn, nwn[...]) + nbn[...]

    out_shape = [jax.ShapeDtypeStruct((_NG, _D), jnp.float32)]
    if has_next:
        out_shape += [jax.ShapeDtypeStruct((_N, _D), jnp.float32)] * 4
        args = (x, u, b2d, s2p, cp, wn3, bn3v) + n2w + gw + nextw
    else:
        args = (x, u, b2d, s2p, cp, wn3, bn3v) + n2w + gw
    res = pl.pallas_call(
        body, out_shape=out_shape,
        compiler_params=pltpu.CompilerParams(
            vmem_limit_bytes=100 * 1024 * 1024),
        interpret=_INTERPRET)(*args)
    return res if has_next else res[0]


def kernel(x, edge_attr, u, params, edge_index, batch):
    f32 = jnp.float32
    row4 = edge_index[0].reshape(_NW, _NCH // _SLOTS, _SLOTS, _GC)
    col4 = edge_index[1].reshape(_NW, _NCH // _SLOTS, _SLOTS, _GC)
    col4s = edge_index[1].reshape(_NW, _SNCH // _SLOTS, _SLOTS, _SC_)
    b2d = batch.reshape(_N, 1)
    znd = jnp.zeros((_NP, _D), f32)
    onesd = jnp.ones((_SC_, _D), f32)
    cp = _sc_counts(col4s, znd, onesd)

    def vec(v):
        return v.reshape(1, -1)

    xl, ul = x, u
    carry = None
    w3p = b3p = None
    nxt = None
    for l in range(3):
        lp = params[l]
        (w1, b1, g1, be1), (w2, b2, g2, be2), (w3, b3) = lp['edge']
        (wn1, bn1, gn1, ben1), (wn2, bn2, gn2, ben2), (wn3, bn3) = lp['node1']
        de = 16 if l == 0 else _D
        w1s, w1d = w1[0:_D], w1[_D:2 * _D]
        w1e, w1u = w1[2 * _D:2 * _D + de], w1[2 * _D + de:]
        if l == 0:
            a, bt, p = _prep(xl, ul, b2d, w1s, w1d, w1u, vec(b1),
                             wn1[:_D], vec(bn1))
            cin, wc, cv1 = edge_attr, w1e, jnp.zeros((1, _D), f32)
        else:
            a, bt, p = nxt
            cin = carry
            wc = w3p @ w1e
            cv1 = vec(b3p @ w1e)
        gth, gth2 = _sc_gather(a, bt, p, row4, col4)
        z1, st1 = _tc1(gth, cin, wc, cv1)
        z2, st2 = _tc2(z1, st1, vec(g1), vec(be1), w2, vec(b2))
        w3n = w3 @ wn1[_D:]
        cv2 = vec(b3 @ wn1[_D:])
        e2, h1, st3 = _tc3(z2, gth2, st2, vec(g2), vec(be2), w3n, cv2, l < 2)
        z4, st4 = _tc4(h1, st3, vec(gn1), vec(ben1), wn2, vec(bn2))
        h2 = _tc5(z4, st4, vec(gn2), vec(ben2))
        s2p = _sc_scatter(h2, col4s, znd)
        (wa1, ba1, ga1, bea1), (wa2, ba2, ga2, bea2), (wa3, ba3) = lp['node2']
        (wg1, bg1, gg1, beg1), (wg2, bg2, gg2, beg2), (wg3, bg3) = lp['global']
        n2w = (wa1[:_D], wa1[_D:2 * _D], wa1[2 * _D:], vec(ba1), vec(ga1),
               vec(bea1), wa2, vec(ba2), vec(ga2), vec(bea2), wa3, vec(ba3))
        gw = (wg1[:_D], wg1[_D:], vec(bg1), vec(gg1), vec(beg1),
              wg2, vec(bg2), vec(gg2), vec(beg2), wg3, vec(bg3))
        if l < 2:
            nlp = params[l + 1]
            nw1, nb1 = nlp['edge'][0][0], nlp['edge'][0][1]
            nwn1, nbn1 = nlp['node1'][0][0], nlp['node1'][0][1]
            nextw = (nw1[0:_D], nw1[_D:2 * _D], nw1[3 * _D:], vec(nb1),
                     nwn1[:_D], vec(nbn1))
            un, xn, na, nb_, np_ = _node(xl, ul, b2d, s2p, cp, wn3, vec(bn3),
                                         n2w, gw, nextw)
            nxt = (na, nb_, np_)
            xl, ul = xn, un
            carry = e2
            w3p, b3p = w3, b3
        else:
            return _node(xl, ul, b2d, s2p, cp, wn3, vec(bn3), n2w, gw, None)


# speedup vs baseline: 4.2502x; 1.1498x over previous
import functools

import jax
import jax.numpy as jnp
from jax import lax
from jax.experimental import pallas as pl
from jax.experimental.pallas import tpu as pltpu
from jax.experimental.pallas import tpu_sc as plsc

_E = 320000
_N = 10000
_D = 128
_NG = 64
_NW = 32
_EPW = _E // _NW
_GC = 80
_NCH = _EPW // _GC
_SLOTS = 5
_SC_ = 40
_SNCH = _EPW // _SC_
_NP = 10240
_NPS = _NP // 16
_BE = 8000
_GRID = _E // _BE
_EPS = 1e-5
_SELU_A = 1.6732632423543772
_SELU_S = 1.0507009873554805

_INTERPRET = False


def _selu(t):
    return _SELU_S * jnp.where(t > 0, t, _SELU_A * (jnp.exp(t) - 1.0))


def _mesh():
    return plsc.VectorSubcoreMesh(core_axis_name="c", subcore_axis_name="s")


def _sc_gather(a, b, p, row4, col4):
    @functools.partial(
        pl.kernel,
        out_type=(jax.ShapeDtypeStruct((_E, _D), jnp.float32),
                  jax.ShapeDtypeStruct((_E, _D), jnp.float32)),
        mesh=_mesh(),
        scratch_types=[
            pltpu.VMEM((_SLOTS, _GC), jnp.int32),
            pltpu.VMEM((_SLOTS, _GC), jnp.int32),
            pltpu.VMEM((_SLOTS, _GC, _D), jnp.float32),
            pltpu.VMEM((_SLOTS, _GC, _D), jnp.float32),
            pltpu.SemaphoreType.DMA((_SLOTS,)),
            pltpu.SemaphoreType.DMA((_SLOTS,)),
        ],
    )
    def k(a_h, b_h, p_h, row_h, col_h, g_h, g2_h, ridx, cidx, bg, bp,
          sga, sgp):
        wid = lax.axis_index("s") * 2 + lax.axis_index("c")
        base = wid * _EPW

        def body(m, carry):
            pltpu.sync_copy(row_h.at[wid, m], ridx)
            pltpu.sync_copy(col_h.at[wid, m], cidx)
            da = [pltpu.async_copy(a_h.at[ridx.at[j]], bg.at[j],
                                   sga.at[j]) for j in range(_SLOTS)]
            dp = [pltpu.async_copy(p_h.at[ridx.at[j]], bp.at[j],
                                   sgp.at[j]) for j in range(_SLOTS)]
            db = []
            for j in range(_SLOTS):
                da[j].wait()
                db.append(pltpu.async_copy(b_h.at[cidx.at[j]],
                                           bg.at[j], sga.at[j], add=True))
            dw = []
            for j in range(_SLOTS):
                off = base + (m * _SLOTS + j) * _GC
                dp[j].wait()
                dw.append(pltpu.async_copy(bp.at[j],
                                           g2_h.at[pl.ds(off, _GC)],
                                           sgp.at[j]))
                db[j].wait()
                dw.append(pltpu.async_copy(bg.at[j],
                                           g_h.at[pl.ds(off, _GC)],
                                           sga.at[j]))
            for d in dw:
                d.wait()
            return carry

        lax.fori_loop(0, _NCH // _SLOTS, body, 0)

    return k(a, b, p, row4, col4)


def _sc_scatter(h2, col4s, znd):
    @functools.partial(
        pl.kernel,
        out_type=jax.ShapeDtypeStruct((2, _NP, _D), jnp.float32),
        mesh=_mesh(),
        scratch_types=[
            pltpu.VMEM((_SLOTS, _SC_), jnp.int32),
            pltpu.VMEM((_SLOTS, _SC_, _D), jnp.float32),
            pltpu.VMEM_SHARED((_NP, _D), jnp.float32),
            pltpu.SemaphoreType.DMA((_SLOTS,)),
            pltpu.SemaphoreType.DMA((_SLOTS,)),
        ],
    )
    def k(h_h, col_h, z_h, out_h, cidx, vbuf, shared, sld, sad):
        cid = lax.axis_index("c")
        sid = lax.axis_index("s")
        wid = sid * 2 + cid
        pltpu.sync_copy(z_h.at[pl.ds(sid * _NPS, _NPS)],
                        shared.at[pl.ds(sid * _NPS, _NPS)])
        plsc.subcore_barrier()

        def body(m, carry):
            pltpu.sync_copy(col_h.at[wid, m], cidx)
            dl = [pltpu.async_copy(
                h_h.at[pl.ds(wid * _EPW + (m * _SLOTS + j) * _SC_, _SC_)],
                vbuf.at[j], sld.at[j]) for j in range(_SLOTS)]
            da = []
            for j in range(_SLOTS):
                dl[j].wait()
                da.append(pltpu.async_copy(vbuf.at[j],
                                           shared.at[cidx.at[j]],
                                           sad.at[j], add=True))
            for d in da:
                d.wait()
            return carry

        lax.fori_loop(0, _SNCH // _SLOTS, body, 0)
        plsc.subcore_barrier()
        pltpu.sync_copy(shared.at[pl.ds(sid * _NPS, _NPS)],
                        out_h.at[cid, pl.ds(sid * _NPS, _NPS)])

    return k(h2, col4s, znd)


def _sc_counts(col4s, z16, ones16):
    @functools.partial(
        pl.kernel,
        out_type=jax.ShapeDtypeStruct((2, _NP, _D), jnp.float32),
        mesh=_mesh(),
        scratch_types=[
            pltpu.VMEM((_SLOTS, _SC_), jnp.int32),
            pltpu.VMEM((_SC_, _D), jnp.float32),
            pltpu.VMEM_SHARED((_NP, _D), jnp.float32),
            pltpu.SemaphoreType.DMA((_SLOTS,)),
        ],
    )
    def k(col_h, z_h, ones_h, out_h, cidx, obuf, shared, sad):
        cid = lax.axis_index("c")
        sid = lax.axis_index("s")
        wid = sid * 2 + cid
        pltpu.sync_copy(ones_h, obuf)
        pltpu.sync_copy(z_h.at[pl.ds(sid * _NPS, _NPS)],
                        shared.at[pl.ds(sid * _NPS, _NPS)])
        plsc.subcore_barrier()

        def body(m, carry):
            pltpu.sync_copy(col_h.at[wid, m], cidx)
            da = [pltpu.async_copy(obuf, shared.at[cidx.at[j]],
                                   sad.at[j], add=True)
                  for j in range(_SLOTS)]
            for d in da:
                d.wait()
            return carry

        lax.fori_loop(0, _SNCH // _SLOTS, body, 0)
        plsc.subcore_barrier()
        pltpu.sync_copy(shared.at[pl.ds(sid * _NPS, _NPS)],
                        out_h.at[cid, pl.ds(sid * _NPS, _NPS)])

    return k(col4s, z16, ones16)


def _bspec_e(w):
    return pl.BlockSpec((_BE, w), lambda i: (i, 0))


def _bspec_c(shape):
    return pl.BlockSpec(shape, lambda i: (0,) * len(shape))


def _f32(ref):
    return ref[...].astype(jnp.float32)


def _acc_stats(st_ref, z):
    st = jnp.concatenate([jnp.sum(z, axis=0, keepdims=True),
                          jnp.sum(z * z, axis=0, keepdims=True),
                          jnp.zeros((6, _D), jnp.float32)], axis=0)
    i = pl.program_id(0)

    @pl.when(i == 0)
    def _():
        st_ref[...] = st

    @pl.when(i != 0)
    def _():
        st_ref[...] = st_ref[...] + st


def _affine(st_ref, gm_ref, bt_ref):
    m = st_ref[0:1, :] * (1.0 / _E)
    v = st_ref[1:2, :] * (1.0 / _E) - m * m
    sc = gm_ref[...] * lax.rsqrt(v + _EPS)
    return sc, bt_ref[...] - m * sc


def _tc1(g, cin, wc, cv):
    de = cin.shape[1]

    def body(g_ref, c_ref, w_ref, v_ref, z_ref, st_ref):
        z = (g_ref[...]
             + jnp.dot(_f32(c_ref), w_ref[...],
                       preferred_element_type=jnp.float32)
             + v_ref[...])
        z_ref[...] = z.astype(jnp.bfloat16)
        _acc_stats(st_ref, z)

    return pl.pallas_call(
        body,
        grid=(_GRID,),
        in_specs=[_bspec_e(_D), _bspec_e(de), _bspec_c((de, _D)),
                  _bspec_c((1, _D))],
        out_specs=[_bspec_e(_D), _bspec_c((8, _D))],
        out_shape=[jax.ShapeDtypeStruct((_E, _D), jnp.bfloat16),
                   jax.ShapeDtypeStruct((8, _D), jnp.float32)],
        interpret=_INTERPRET,
    )(g, cin, wc, cv)


def _tc2(z1, st1, gm, bt, w, b):
    def body(z_ref, st_ref, gm_ref, bt_ref, w_ref, b_ref, o_ref, so_ref):
        sc, sh = _affine(st_ref, gm_ref, bt_ref)
        e1 = _selu(_f32(z_ref) * sc + sh)
        z2 = jnp.dot(e1, w_ref[...],
                     preferred_element_type=jnp.float32) + b_ref[...]
        o_ref[...] = z2.astype(jnp.bfloat16)
        _acc_stats(so_ref, z2)

    return pl.pallas_call(
        body,
        grid=(_GRID,),
        in_specs=[_bspec_e(_D), _bspec_c((8, _D)), _bspec_c((1, _D)),
                  _bspec_c((1, _D)), _bspec_c((_D, _D)), _bspec_c((1, _D))],
        out_specs=[_bspec_e(_D), _bspec_c((8, _D))],
        out_shape=[jax.ShapeDtypeStruct((_E, _D), jnp.bfloat16),
                   jax.ShapeDtypeStruct((8, _D), jnp.float32)],
        interpret=_INTERPRET,
    )(z1, st1, gm, bt, w, b)


def _tc3(z2, g2, st2, gm, bt, w3n, cv2, write_e2):
    def body_full(z_ref, g2_ref, st_ref, gm_ref, bt_ref, w_ref, v_ref,
                  e2_ref, h1_ref, so_ref):
        sc, sh = _affine(st_ref, gm_ref, bt_ref)
        e2 = _selu(_f32(z_ref) * sc + sh)
        e2_ref[...] = e2.astype(jnp.bfloat16)
        h1 = (g2_ref[...]
              + jnp.dot(e2, w_ref[...], preferred_element_type=jnp.float32)
              + v_ref[...])
        h1_ref[...] = h1.astype(jnp.bfloat16)
        _acc_stats(so_ref, h1)

    def body_nocarry(z_ref, g2_ref, st_ref, gm_ref, bt_ref, w_ref, v_ref,
                     h1_ref, so_ref):
        sc, sh = _affine(st_ref, gm_ref, bt_ref)
        e2 = _selu(_f32(z_ref) * sc + sh)
        h1 = (g2_ref[...]
              + jnp.dot(e2, w_ref[...], preferred_element_type=jnp.float32)
              + v_ref[...])
        h1_ref[...] = h1.astype(jnp.bfloat16)
        _acc_stats(so_ref, h1)

    in_specs = [_bspec_e(_D), _bspec_e(_D), _bspec_c((8, _D)),
                _bspec_c((1, _D)), _bspec_c((1, _D)), _bspec_c((_D, _D)),
                _bspec_c((1, _D))]
    if write_e2:
        return pl.pallas_call(
            body_full,
            grid=(_GRID,),
            in_specs=in_specs,
            out_specs=[_bspec_e(_D), _bspec_e(_D), _bspec_c((8, _D))],
            out_shape=[jax.ShapeDtypeStruct((_E, _D), jnp.bfloat16),
                       jax.ShapeDtypeStruct((_E, _D), jnp.bfloat16),
                       jax.ShapeDtypeStruct((8, _D), jnp.float32)],
            interpret=_INTERPRET,
        )(z2, g2, st2, gm, bt, w3n, cv2)
    h1, st3 = pl.pallas_call(
        body_nocarry,
        grid=(_GRID,),
        in_specs=in_specs,
        out_specs=[_bspec_e(_D), _bspec_c((8, _D))],
        out_shape=[jax.ShapeDtypeStruct((_E, _D), jnp.bfloat16),
                   jax.ShapeDtypeStruct((8, _D), jnp.float32)],
        interpret=_INTERPRET,
    )(z2, g2, st2, gm, bt, w3n, cv2)
    return None, h1, st3


def _tc4(h1, st3, gm, bt, w, b):
    def body(h_ref, st_ref, gm_ref, bt_ref, w_ref, b_ref, o_ref, so_ref):
        sc, sh = _affine(st_ref, gm_ref, bt_ref)
        n1 = _selu(_f32(h_ref) * sc + sh)
        z4 = jnp.dot(n1, w_ref[...],
                     preferred_element_type=jnp.float32) + b_ref[...]
        o_ref[...] = z4.astype(jnp.bfloat16)
        _acc_stats(so_ref, z4)

    return pl.pallas_call(
        body,
        grid=(_GRID,),
        in_specs=[_bspec_e(_D), _bspec_c((8, _D)), _bspec_c((1, _D)),
                  _bspec_c((1, _D)), _bspec_c((_D, _D)), _bspec_c((1, _D))],
        out_specs=[_bspec_e(_D), _bspec_c((8, _D))],
        out_shape=[jax.ShapeDtypeStruct((_E, _D), jnp.bfloat16),
                   jax.ShapeDtypeStruct((8, _D), jnp.float32)],
        interpret=_INTERPRET,
    )(h1, st3, gm, bt, w, b)


def _tc5(z4, st4, gm, bt):
    def body(z_ref, st_ref, gm_ref, bt_ref, o_ref):
        sc, sh = _affine(st_ref, gm_ref, bt_ref)
        o_ref[...] = _selu(_f32(z_ref) * sc + sh)

    return pl.pallas_call(
        body,
        grid=(_GRID,),
        in_specs=[_bspec_e(_D), _bspec_c((8, _D)), _bspec_c((1, _D)),
                  _bspec_c((1, _D))],
        out_specs=[_bspec_e(_D)],
        out_shape=[jax.ShapeDtypeStruct((_E, _D), jnp.float32)],
        interpret=_INTERPRET,
    )(z4, st4, gm, bt)[0]


def _onehot(b_ref):
    return (b_ref[...] == lax.broadcasted_iota(
        jnp.int32, (_N, _NG), 1)).astype(jnp.float32)


def _prep(x, u, b2d, wsrc, wdst, wu, b1v, wnx, bnv):
    def body(x_ref, u_ref, b_ref, ws, wd, wu_, b1_, wn_, bn_,
             a_ref, bb_ref, p_ref):
        xx = x_ref[...]
        oh = _onehot(b_ref)
        up = jnp.dot(u_ref[...], wu_[...], preferred_element_type=jnp.float32)
        a_ref[...] = (jnp.dot(xx, ws[...], preferred_element_type=jnp.float32)
                      + jnp.dot(oh, up, preferred_element_type=jnp.float32)
                      + b1_[...])
        bb_ref[...] = jnp.dot(xx, wd[...], preferred_element_type=jnp.float32)
        p_ref[...] = jnp.dot(xx, wn_[...],
                             preferred_element_type=jnp.float32) + bn_[...]

    return pl.pallas_call(
        body,
        out_shape=[jax.ShapeDtypeStruct((_N, _D), jnp.float32)] * 3,
        interpret=_INTERPRET,
    )(x, u, b2d, wsrc, wdst, wu, b1v, wnx, bnv)


def _bn_rows(t, gm, bt):
    m = jnp.mean(t, axis=0, keepdims=True)
    v = jnp.mean((t - m) ** 2, axis=0, keepdims=True)
    return (t - m) * lax.rsqrt(v + _EPS) * gm + bt


def _node(x, u, b2d, s2p, cp, wn3, bn3v, n2w, gw, nextw):
    has_next = nextw is not None

    def body(*refs):
        (x_ref, u_ref, b_ref, s_ref, c_ref, wn3_, bn3_,
         wax, wag, wau, ba1, ga1, bea1, wa2, ba2, ga2, bea2, wa3, ba3,
         wg1u, wg1n, bg1, gg1, beg1, wg2, bg2, gg2, beg2, wg3, bg3) = refs[:30]
        if has_next:
            nws, nwd, nwu, nb1, nwn, nbn = refs[30:36]
            un_ref, xn_ref, a_ref, bb_ref, p_ref = refs[36:]
        else:
            un_ref = refs[30]

        def dot(a, b):
            return jnp.dot(a, b, preferred_element_type=jnp.float32)

        s2 = s_ref[0, 0:_N, :] + s_ref[1, 0:_N, :]
        cc = c_ref[0, 0:_N, 0:1] + c_ref[1, 0:_N, 0:1]
        agg = (dot(s2, wn3_[...]) + cc * bn3_[...]) / jnp.maximum(cc, 1.0)
        oh = _onehot(b_ref)
        xx = x_ref[...]
        uu = u_ref[...]
        t = (dot(xx, wax[...]) + dot(agg, wag[...])
             + dot(oh, dot(uu, wau[...])) + ba1[...])
        t = _selu(_bn_rows(t, ga1[...], bea1[...]))
        t = dot(t, wa2[...]) + ba2[...]
        t = _selu(_bn_rows(t, ga2[...], bea2[...]))
        xn = dot(t, wa3[...]) + ba3[...]

        nc = lax.dot_general(oh, jnp.full((_N, 1), 1.0, jnp.float32),
                             (((0,), (0,)), ((), ())),
                             preferred_element_type=jnp.float32)
        ns = lax.dot_general(oh, xn, (((0,), (0,)), ((), ())),
                             preferred_element_type=jnp.float32)
        nmean = ns / jnp.maximum(nc, 1.0)
        g = dot(uu, wg1u[...]) + dot(nmean, wg1n[...]) + bg1[...]
        g = _selu(_bn_rows(g, gg1[...], beg1[...]))
        g = dot(g, wg2[...]) + bg2[...]
        g = _selu(_bn_rows(g, gg2[...], beg2[...]))
        un = dot(g, wg3[...]) + bg3[...]
        un_ref[...] = un
        if has_next:
            xn_ref[...] = xn
            a_ref[...] = (dot(xn, nws[...]) + dot(oh, dot(un, nwu[...]))
                          + nb1[...])
            bb_ref[...] = dot(xn, nwd[...])
            p_ref[...] = dot(xn, nwn[...]) + nbn[...]

    out_shape = [jax.ShapeDtypeStruct((_NG, _D), jnp.float32)]
    if has_next:
        out_shape += [jax.ShapeDtypeStruct((_N, _D), jnp.float32)] * 4
        args = (x, u, b2d, s2p, cp, wn3, bn3v) + n2w + gw + nextw
    else:
        args = (x, u, b2d, s2p, cp, wn3, bn3v) + n2w + gw
    res = pl.pallas_call(
        body, out_shape=out_shape,
        compiler_params=pltpu.CompilerParams(
            vmem_limit_bytes=100 * 1024 * 1024),
        interpret=_INTERPRET)(*args)
    return res if has_next else res[0]


def kernel(x, edge_attr, u, params, edge_index, batch):
    f32 = jnp.float32
    row4 = edge_index[0].reshape(_NW, _NCH // _SLOTS, _SLOTS, _GC)
    col4 = edge_index[1].reshape(_NW, _NCH // _SLOTS, _SLOTS, _GC)
    col4s = edge_index[1].reshape(_NW, _SNCH // _SLOTS, _SLOTS, _SC_)
    b2d = batch.reshape(_N, 1)
    znd = jnp.zeros((_NP, _D), f32)
    onesd = jnp.ones((_SC_, _D), f32)
    cp = _sc_counts(col4s, znd, onesd)

    def vec(v):
        return v.reshape(1, -1)

    xl, ul = x, u
    carry = None
    w3p = b3p = None
    nxt = None
    for l in range(3):
        lp = params[l]
        (w1, b1, g1, be1), (w2, b2, g2, be2), (w3, b3) = lp['edge']
        (wn1, bn1, gn1, ben1), (wn2, bn2, gn2, ben2), (wn3, bn3) = lp['node1']
        de = 16 if l == 0 else _D
        w1s, w1d = w1[0:_D], w1[_D:2 * _D]
        w1e, w1u = w1[2 * _D:2 * _D + de], w1[2 * _D + de:]
        if l == 0:
            a, bt, p = _prep(xl, ul, b2d, w1s, w1d, w1u, vec(b1),
                             wn1[:_D], vec(bn1))
            cin, wc, cv1 = edge_attr, w1e, jnp.zeros((1, _D), f32)
        else:
            a, bt, p = nxt
            cin = carry
            wc = w3p @ w1e
            cv1 = vec(b3p @ w1e)
        gth, gth2 = _sc_gather(a, bt, p, row4, col4)
        z1, st1 = _tc1(gth, cin, wc, cv1)
        z2, st2 = _tc2(z1, st1, vec(g1), vec(be1), w2, vec(b2))
        w3n = w3 @ wn1[_D:]
        cv2 = vec(b3 @ wn1[_D:])
        e2, h1, st3 = _tc3(z2, gth2, st2, vec(g2), vec(be2), w3n, cv2, l < 2)
        z4, st4 = _tc4(h1, st3, vec(gn1), vec(ben1), wn2, vec(bn2))
        h2 = _tc5(z4, st4, vec(gn2), vec(ben2))
        s2p = _sc_scatter(h2, col4s, znd)
        (wa1, ba1, ga1, bea1), (wa2, ba2, ga2, bea2), (wa3, ba3) = lp['node2']
        (wg1, bg1, gg1, beg1), (wg2, bg2, gg2, beg2), (wg3, bg3) = lp['global']
        n2w = (wa1[:_D], wa1[_D:2 * _D], wa1[2 * _D:], vec(ba1), vec(ga1),
               vec(bea1), wa2, vec(ba2), vec(ga2), vec(bea2), wa3, vec(ba3))
        gw = (wg1[:_D], wg1[_D:], vec(bg1), vec(gg1), vec(beg1),
              wg2, vec(bg2), vec(gg2), vec(beg2), wg3, vec(bg3))
        if l < 2:
            nlp = params[l + 1]
            nw1, nb1 = nlp['edge'][0][0], nlp['edge'][0][1]
            nwn1, nbn1 = nlp['node1'][0][0], nlp['node1'][0][1]
            nextw = (nw1[0:_D], nw1[_D:2 * _D], nw1[3 * _D:], vec(nb1),
                     nwn1[:_D], vec(nbn1))
            un, xn, na, nb_, np_ = _node(xl, ul, b2d, s2p, cp, wn3, vec(bn3),
                                         n2w, gw, nextw)
            nxt = (na, nb_, np_)
            xl, ul = xn, un
            carry = e2
            w3p, b3p = w3, b3
        else:
            return _node(xl, ul, b2d, s2p, cp, wn3, vec(bn3), n2w, gw, None)


# speedup vs baseline: 4.3800x; 1.0305x over previous
import functools

import jax
import jax.numpy as jnp
from jax import lax
from jax.experimental import pallas as pl
from jax.experimental.pallas import tpu as pltpu
from jax.experimental.pallas import tpu_sc as plsc

_E = 320000
_N = 10000
_D = 128
_NG = 64
_NW = 32
_EPW = _E // _NW
_GC = 80
_NCH = _EPW // _GC
_SLOTS = 5
_SC_ = 40
_SNCH = _EPW // _SC_
_NP = 10240
_NPS = _NP // 16
_BE = 16000
_GRID = _E // _BE
_EPS = 1e-5
_SELU_A = 1.6732632423543772
_SELU_S = 1.0507009873554805

_INTERPRET = False


def _selu(t):
    return _SELU_S * jnp.where(t > 0, t, _SELU_A * (jnp.exp(t) - 1.0))


def _mesh():
    return plsc.VectorSubcoreMesh(core_axis_name="c", subcore_axis_name="s")


def _sc_gather(a, b, p, row4, col4):
    @functools.partial(
        pl.kernel,
        out_type=(jax.ShapeDtypeStruct((_E, _D), jnp.float32),
                  jax.ShapeDtypeStruct((_E, _D), jnp.float32)),
        mesh=_mesh(),
        scratch_types=[
            pltpu.VMEM((_SLOTS, _GC), jnp.int32),
            pltpu.VMEM((_SLOTS, _GC), jnp.int32),
            pltpu.VMEM((_SLOTS, _GC, _D), jnp.float32),
            pltpu.VMEM((_SLOTS, _GC, _D), jnp.float32),
            pltpu.SemaphoreType.DMA((_SLOTS,)),
            pltpu.SemaphoreType.DMA((_SLOTS,)),
        ],
    )
    def k(a_h, b_h, p_h, row_h, col_h, g_h, g2_h, ridx, cidx, bg, bp,
          sga, sgp):
        wid = lax.axis_index("s") * 2 + lax.axis_index("c")
        base = wid * _EPW

        def body(m, carry):
            pltpu.sync_copy(row_h.at[wid, m], ridx)
            pltpu.sync_copy(col_h.at[wid, m], cidx)
            da = [pltpu.async_copy(a_h.at[ridx.at[j]], bg.at[j],
                                   sga.at[j]) for j in range(_SLOTS)]
            dp = [pltpu.async_copy(p_h.at[ridx.at[j]], bp.at[j],
                                   sgp.at[j]) for j in range(_SLOTS)]
            db = []
            for j in range(_SLOTS):
                da[j].wait()
                db.append(pltpu.async_copy(b_h.at[cidx.at[j]],
                                           bg.at[j], sga.at[j], add=True))
            dw = []
            for j in range(_SLOTS):
                off = base + (m * _SLOTS + j) * _GC
                dp[j].wait()
                dw.append(pltpu.async_copy(bp.at[j],
                                           g2_h.at[pl.ds(off, _GC)],
                                           sgp.at[j]))
                db[j].wait()
                dw.append(pltpu.async_copy(bg.at[j],
                                           g_h.at[pl.ds(off, _GC)],
                                           sga.at[j]))
            for d in dw:
                d.wait()
            return carry

        lax.fori_loop(0, _NCH // _SLOTS, body, 0)

    return k(a, b, p, row4, col4)


def _sc_scatter(h2, col4s, znd):
    @functools.partial(
        pl.kernel,
        out_type=jax.ShapeDtypeStruct((2, _NP, _D), jnp.float32),
        mesh=_mesh(),
        scratch_types=[
            pltpu.VMEM((_SLOTS, _SC_), jnp.int32),
            pltpu.VMEM((_SLOTS, _SC_, _D), jnp.float32),
            pltpu.VMEM_SHARED((_NP, _D), jnp.float32),
            pltpu.SemaphoreType.DMA((_SLOTS,)),
            pltpu.SemaphoreType.DMA((_SLOTS,)),
        ],
    )
    def k(h_h, col_h, z_h, out_h, cidx, vbuf, shared, sld, sad):
        cid = lax.axis_index("c")
        sid = lax.axis_index("s")
        wid = sid * 2 + cid
        pltpu.sync_copy(z_h.at[pl.ds(sid * _NPS, _NPS)],
                        shared.at[pl.ds(sid * _NPS, _NPS)])
        plsc.subcore_barrier()

        def body(m, carry):
            pltpu.sync_copy(col_h.at[wid, m], cidx)
            dl = [pltpu.async_copy(
                h_h.at[pl.ds(wid * _EPW + (m * _SLOTS + j) * _SC_, _SC_)],
                vbuf.at[j], sld.at[j]) for j in range(_SLOTS)]
            da = []
            for j in range(_SLOTS):
                dl[j].wait()
                da.append(pltpu.async_copy(vbuf.at[j],
                                           shared.at[cidx.at[j]],
                                           sad.at[j], add=True))
            for d in da:
                d.wait()
            return carry

        lax.fori_loop(0, _SNCH // _SLOTS, body, 0)
        plsc.subcore_barrier()
        pltpu.sync_copy(shared.at[pl.ds(sid * _NPS, _NPS)],
                        out_h.at[cid, pl.ds(sid * _NPS, _NPS)])

    return k(h2, col4s, znd)


def _sc_counts(col4s, z16, ones16):
    @functools.partial(
        pl.kernel,
        out_type=jax.ShapeDtypeStruct((2, _NP, _D), jnp.float32),
        mesh=_mesh(),
        scratch_types=[
            pltpu.VMEM((_SLOTS, _SC_), jnp.int32),
            pltpu.VMEM((_SC_, _D), jnp.float32),
            pltpu.VMEM_SHARED((_NP, _D), jnp.float32),
            pltpu.SemaphoreType.DMA((_SLOTS,)),
        ],
    )
    def k(col_h, z_h, ones_h, out_h, cidx, obuf, shared, sad):
        cid = lax.axis_index("c")
        sid = lax.axis_index("s")
        wid = sid * 2 + cid
        pltpu.sync_copy(ones_h, obuf)
        pltpu.sync_copy(z_h.at[pl.ds(sid * _NPS, _NPS)],
                        shared.at[pl.ds(sid * _NPS, _NPS)])
        plsc.subcore_barrier()

        def body(m, carry):
            pltpu.sync_copy(col_h.at[wid, m], cidx)
            da = [pltpu.async_copy(obuf, shared.at[cidx.at[j]],
                                   sad.at[j], add=True)
                  for j in range(_SLOTS)]
            for d in da:
                d.wait()
            return carry

        lax.fori_loop(0, _SNCH // _SLOTS, body, 0)
        plsc.subcore_barrier()
        pltpu.sync_copy(shared.at[pl.ds(sid * _NPS, _NPS)],
                        out_h.at[cid, pl.ds(sid * _NPS, _NPS)])

    return k(col4s, z16, ones16)


def _bspec_e(w):
    return pl.BlockSpec((_BE, w), lambda i: (i, 0))


def _bspec_c(shape):
    return pl.BlockSpec(shape, lambda i: (0,) * len(shape))


def _f32(ref):
    return ref[...].astype(jnp.float32)


def _acc_stats(st_ref, z):
    st = jnp.concatenate([jnp.sum(z, axis=0, keepdims=True),
                          jnp.sum(z * z, axis=0, keepdims=True),
                          jnp.zeros((6, _D), jnp.float32)], axis=0)
    i = pl.program_id(0)

    @pl.when(i == 0)
    def _():
        st_ref[...] = st

    @pl.when(i != 0)
    def _():
        st_ref[...] = st_ref[...] + st


def _affine(st_ref, gm_ref, bt_ref):
    m = st_ref[0:1, :] * (1.0 / _E)
    v = st_ref[1:2, :] * (1.0 / _E) - m * m
    sc = gm_ref[...] * lax.rsqrt(v + _EPS)
    return sc, bt_ref[...] - m * sc


def _tc1(g, cin, wc, cv):
    de = cin.shape[1]

    def body(g_ref, c_ref, w_ref, v_ref, z_ref, st_ref):
        z = (g_ref[...]
             + jnp.dot(_f32(c_ref), w_ref[...],
                       preferred_element_type=jnp.float32)
             + v_ref[...])
        z_ref[...] = z.astype(jnp.bfloat16)
        _acc_stats(st_ref, z)

    return pl.pallas_call(
        body,
        grid=(_GRID,),
        in_specs=[_bspec_e(_D), _bspec_e(de), _bspec_c((de, _D)),
                  _bspec_c((1, _D))],
        out_specs=[_bspec_e(_D), _bspec_c((8, _D))],
        out_shape=[jax.ShapeDtypeStruct((_E, _D), jnp.bfloat16),
                   jax.ShapeDtypeStruct((8, _D), jnp.float32)],
        interpret=_INTERPRET,
    )(g, cin, wc, cv)


def _tc2(z1, st1, gm, bt, w, b):
    def body(z_ref, st_ref, gm_ref, bt_ref, w_ref, b_ref, o_ref, so_ref):
        sc, sh = _affine(st_ref, gm_ref, bt_ref)
        e1 = _selu(_f32(z_ref) * sc + sh)
        z2 = jnp.dot(e1, w_ref[...],
                     preferred_element_type=jnp.float32) + b_ref[...]
        o_ref[...] = z2.astype(jnp.bfloat16)
        _acc_stats(so_ref, z2)

    return pl.pallas_call(
        body,
        grid=(_GRID,),
        in_specs=[_bspec_e(_D), _bspec_c((8, _D)), _bspec_c((1, _D)),
                  _bspec_c((1, _D)), _bspec_c((_D, _D)), _bspec_c((1, _D))],
        out_specs=[_bspec_e(_D), _bspec_c((8, _D))],
        out_shape=[jax.ShapeDtypeStruct((_E, _D), jnp.bfloat16),
                   jax.ShapeDtypeStruct((8, _D), jnp.float32)],
        interpret=_INTERPRET,
    )(z1, st1, gm, bt, w, b)


def _tc3(z2, g2, st2, gm, bt, w3n, cv2, write_e2):
    def body_full(z_ref, g2_ref, st_ref, gm_ref, bt_ref, w_ref, v_ref,
                  e2_ref, h1_ref, so_ref):
        sc, sh = _affine(st_ref, gm_ref, bt_ref)
        e2 = _selu(_f32(z_ref) * sc + sh)
        e2_ref[...] = e2.astype(jnp.bfloat16)
        h1 = (g2_ref[...]
              + jnp.dot(e2, w_ref[...], preferred_element_type=jnp.float32)
              + v_ref[...])
        h1_ref[...] = h1.astype(jnp.bfloat16)
        _acc_stats(so_ref, h1)

    def body_nocarry(z_ref, g2_ref, st_ref, gm_ref, bt_ref, w_ref, v_ref,
                     h1_ref, so_ref):
        sc, sh = _affine(st_ref, gm_ref, bt_ref)
        e2 = _selu(_f32(z_ref) * sc + sh)
        h1 = (g2_ref[...]
              + jnp.dot(e2, w_ref[...], preferred_element_type=jnp.float32)
              + v_ref[...])
        h1_ref[...] = h1.astype(jnp.bfloat16)
        _acc_stats(so_ref, h1)

    in_specs = [_bspec_e(_D), _bspec_e(_D), _bspec_c((8, _D)),
                _bspec_c((1, _D)), _bspec_c((1, _D)), _bspec_c((_D, _D)),
                _bspec_c((1, _D))]
    if write_e2:
        return pl.pallas_call(
            body_full,
            grid=(_GRID,),
            in_specs=in_specs,
            out_specs=[_bspec_e(_D), _bspec_e(_D), _bspec_c((8, _D))],
            out_shape=[jax.ShapeDtypeStruct((_E, _D), jnp.bfloat16),
                       jax.ShapeDtypeStruct((_E, _D), jnp.bfloat16),
                       jax.ShapeDtypeStruct((8, _D), jnp.float32)],
            interpret=_INTERPRET,
        )(z2, g2, st2, gm, bt, w3n, cv2)
    h1, st3 = pl.pallas_call(
        body_nocarry,
        grid=(_GRID,),
        in_specs=in_specs,
        out_specs=[_bspec_e(_D), _bspec_c((8, _D))],
        out_shape=[jax.ShapeDtypeStruct((_E, _D), jnp.bfloat16),
                   jax.ShapeDtypeStruct((8, _D), jnp.float32)],
        interpret=_INTERPRET,
    )(z2, g2, st2, gm, bt, w3n, cv2)
    return None, h1, st3


def _tc4(h1, st3, gm, bt, w, b):
    def body(h_ref, st_ref, gm_ref, bt_ref, w_ref, b_ref, o_ref, so_ref):
        sc, sh = _affine(st_ref, gm_ref, bt_ref)
        n1 = _selu(_f32(h_ref) * sc + sh)
        z4 = jnp.dot(n1, w_ref[...],
                     preferred_element_type=jnp.float32) + b_ref[...]
        o_ref[...] = z4.astype(jnp.bfloat16)
        _acc_stats(so_ref, z4)

    return pl.pallas_call(
        body,
        grid=(_GRID,),
        in_specs=[_bspec_e(_D), _bspec_c((8, _D)), _bspec_c((1, _D)),
                  _bspec_c((1, _D)), _bspec_c((_D, _D)), _bspec_c((1, _D))],
        out_specs=[_bspec_e(_D), _bspec_c((8, _D))],
        out_shape=[jax.ShapeDtypeStruct((_E, _D), jnp.bfloat16),
                   jax.ShapeDtypeStruct((8, _D), jnp.float32)],
        interpret=_INTERPRET,
    )(h1, st3, gm, bt, w, b)


def _tc5(z4, st4, gm, bt):
    def body(z_ref, st_ref, gm_ref, bt_ref, o_ref):
        sc, sh = _affine(st_ref, gm_ref, bt_ref)
        o_ref[...] = _selu(_f32(z_ref) * sc + sh)

    return pl.pallas_call(
        body,
        grid=(_GRID,),
        in_specs=[_bspec_e(_D), _bspec_c((8, _D)), _bspec_c((1, _D)),
                  _bspec_c((1, _D))],
        out_specs=[_bspec_e(_D)],
        out_shape=[jax.ShapeDtypeStruct((_E, _D), jnp.float32)],
        interpret=_INTERPRET,
    )(z4, st4, gm, bt)[0]


def _onehot(b_ref):
    return (b_ref[...] == lax.broadcasted_iota(
        jnp.int32, (_N, _NG), 1)).astype(jnp.float32)


def _prep(x, u, b2d, wsrc, wdst, wu, b1v, wnx, bnv):
    def body(x_ref, u_ref, b_ref, ws, wd, wu_, b1_, wn_, bn_,
             a_ref, bb_ref, p_ref):
        xx = x_ref[...]
        oh = _onehot(b_ref)
        up = jnp.dot(u_ref[...], wu_[...], preferred_element_type=jnp.float32)
        a_ref[...] = (jnp.dot(xx, ws[...], preferred_element_type=jnp.float32)
                      + jnp.dot(oh, up, preferred_element_type=jnp.float32)
                      + b1_[...])
        bb_ref[...] = jnp.dot(xx, wd[...], preferred_element_type=jnp.float32)
        p_ref[...] = jnp.dot(xx, wn_[...],
                             preferred_element_type=jnp.float32) + bn_[...]

    return pl.pallas_call(
        body,
        out_shape=[jax.ShapeDtypeStruct((_N, _D), jnp.float32)] * 3,
        interpret=_INTERPRET,
    )(x, u, b2d, wsrc, wdst, wu, b1v, wnx, bnv)


def _bn_rows(t, gm, bt):
    m = jnp.mean(t, axis=0, keepdims=True)
    v = jnp.mean((t - m) ** 2, axis=0, keepdims=True)
    return (t - m) * lax.rsqrt(v + _EPS) * gm + bt


def _node(x, u, b2d, s2p, cp, wn3, bn3v, n2w, gw, nextw):
    has_next = nextw is not None

    def body(*refs):
        (x_ref, u_ref, b_ref, s_ref, c_ref, wn3_, bn3_,
         wax, wag, wau, ba1, ga1, bea1, wa2, ba2, ga2, bea2, wa3, ba3,
         wg1u, wg1n, bg1, gg1, beg1, wg2, bg2, gg2, beg2, wg3, bg3) = refs[:30]
        if has_next:
            nws, nwd, nwu, nb1, nwn, nbn = refs[30:36]
            un_ref, xn_ref, a_ref, bb_ref, p_ref = refs[36:]
        else:
            un_ref = refs[30]

        def dot(a, b):
            return jnp.dot(a, b, preferred_element_type=jnp.float32)

        s2 = s_ref[0, 0:_N, :] + s_ref[1, 0:_N, :]
        cc = c_ref[0, 0:_N, 0:1] + c_ref[1, 0:_N, 0:1]
        agg = (dot(s2, wn3_[...]) + cc * bn3_[...]) / jnp.maximum(cc, 1.0)
        oh = _onehot(b_ref)
        xx = x_ref[...]
        uu = u_ref[...]
        t = (dot(xx, wax[...]) + dot(agg, wag[...])
             + dot(oh, dot(uu, wau[...])) + ba1[...])
        t = _selu(_bn_rows(t, ga1[...], bea1[...]))
        t = dot(t, wa2[...]) + ba2[...]
        t = _selu(_bn_rows(t, ga2[...], bea2[...]))
        xn = dot(t, wa3[...]) + ba3[...]

        nc = lax.dot_general(oh, jnp.full((_N, 1), 1.0, jnp.float32),
                             (((0,), (0,)), ((), ())),
                             preferred_element_type=jnp.float32)
        ns = lax.dot_general(oh, xn, (((0,), (0,)), ((), ())),
                             preferred_element_type=jnp.float32)
        nmean = ns / jnp.maximum(nc, 1.0)
        g = dot(uu, wg1u[...]) + dot(nmean, wg1n[...]) + bg1[...]
        g = _selu(_bn_rows(g, gg1[...], beg1[...]))
        g = dot(g, wg2[...]) + bg2[...]
        g = _selu(_bn_rows(g, gg2[...], beg2[...]))
        un = dot(g, wg3[...]) + bg3[...]
        un_ref[...] = un
        if has_next:
            xn_ref[...] = xn
            a_ref[...] = (dot(xn, nws[...]) + dot(oh, dot(un, nwu[...]))
                          + nb1[...])
            bb_ref[...] = dot(xn, nwd[...])
            p_ref[...] = dot(xn, nwn[...]) + nbn[...]

    out_shape = [jax.ShapeDtypeStruct((_NG, _D), jnp.float32)]
    if has_next:
        out_shape += [jax.ShapeDtypeStruct((_N, _D), jnp.float32)] * 4
        args = (x, u, b2d, s2p, cp, wn3, bn3v) + n2w + gw + nextw
    else:
        args = (x, u, b2d, s2p, cp, wn3, bn3v) + n2w + gw
    res = pl.pallas_call(
        body, out_shape=out_shape,
        compiler_params=pltpu.CompilerParams(
            vmem_limit_bytes=100 * 1024 * 1024),
        interpret=_INTERPRET)(*args)
    return res if has_next else res[0]


def kernel(x, edge_attr, u, params, edge_index, batch):
    f32 = jnp.float32
    row4 = edge_index[0].reshape(_NW, _NCH // _SLOTS, _SLOTS, _GC)
    col4 = edge_index[1].reshape(_NW, _NCH // _SLOTS, _SLOTS, _GC)
    col4s = edge_index[1].reshape(_NW, _SNCH // _SLOTS, _SLOTS, _SC_)
    b2d = batch.reshape(_N, 1)
    znd = jnp.zeros((_NP, _D), f32)
    onesd = jnp.ones((_SC_, _D), f32)
    cp = _sc_counts(col4s, znd, onesd)

    def vec(v):
        return v.reshape(1, -1)

    xl, ul = x, u
    carry = None
    w3p = b3p = None
    nxt = None
    for l in range(3):
        lp = params[l]
        (w1, b1, g1, be1), (w2, b2, g2, be2), (w3, b3) = lp['edge']
        (wn1, bn1, gn1, ben1), (wn2, bn2, gn2, ben2), (wn3, bn3) = lp['node1']
        de = 16 if l == 0 else _D
        w1s, w1d = w1[0:_D], w1[_D:2 * _D]
        w1e, w1u = w1[2 * _D:2 * _D + de], w1[2 * _D + de:]
        if l == 0:
            a, bt, p = _prep(xl, ul, b2d, w1s, w1d, w1u, vec(b1),
                             wn1[:_D], vec(bn1))
            cin, wc, cv1 = edge_attr, w1e, jnp.zeros((1, _D), f32)
        else:
            a, bt, p = nxt
            cin = carry
            wc = w3p @ w1e
            cv1 = vec(b3p @ w1e)
        gth, gth2 = _sc_gather(a, bt, p, row4, col4)
        z1, st1 = _tc1(gth, cin, wc, cv1)
        z2, st2 = _tc2(z1, st1, vec(g1), vec(be1), w2, vec(b2))
        w3n = w3 @ wn1[_D:]
        cv2 = vec(b3 @ wn1[_D:])
        e2, h1, st3 = _tc3(z2, gth2, st2, vec(g2), vec(be2), w3n, cv2, l < 2)
        z4, st4 = _tc4(h1, st3, vec(gn1), vec(ben1), wn2, vec(bn2))
        h2 = _tc5(z4, st4, vec(gn2), vec(ben2))
        s2p = _sc_scatter(h2, col4s, znd)
        (wa1, ba1, ga1, bea1), (wa2, ba2, ga2, bea2), (wa3, ba3) = lp['node2']
        (wg1, bg1, gg1, beg1), (wg2, bg2, gg2, beg2), (wg3, bg3) = lp['global']
        n2w = (wa1[:_D], wa1[_D:2 * _D], wa1[2 * _D:], vec(ba1), vec(ga1),
               vec(bea1), wa2, vec(ba2), vec(ga2), vec(bea2), wa3, vec(ba3))
        gw = (wg1[:_D], wg1[_D:], vec(bg1), vec(gg1), vec(beg1),
              wg2, vec(bg2), vec(gg2), vec(beg2), wg3, vec(bg3))
        if l < 2:
            nlp = params[l + 1]
            nw1, nb1 = nlp['edge'][0][0], nlp['edge'][0][1]
            nwn1, nbn1 = nlp['node1'][0][0], nlp['node1'][0][1]
            nextw = (nw1[0:_D], nw1[_D:2 * _D], nw1[3 * _D:], vec(nb1),
                     nwn1[:_D], vec(nbn1))
            un, xn, na, nb_, np_ = _node(xl, ul, b2d, s2p, cp, wn3, vec(bn3),
                                         n2w, gw, nextw)
            nxt = (na, nb_, np_)
            xl, ul = xn, un
            carry = e2
            w3p, b3p = w3, b3
        else:
            return _node(xl, ul, b2d, s2p, cp, wn3, vec(bn3), n2w, gw, None)


# speedup vs baseline: 4.3802x; 1.0000x over previous
import functools

import jax
import jax.numpy as jnp
from jax import lax
from jax.experimental import pallas as pl
from jax.experimental.pallas import tpu as pltpu
from jax.experimental.pallas import tpu_sc as plsc

_E = 320000
_N = 10000
_D = 128
_NG = 64
_NW = 32
_EPW = _E // _NW
_GC = 80
_NCH = _EPW // _GC
_SLOTS = 5
_SC_ = 40
_SNCH = _EPW // _SC_
_NP = 10240
_NPS = _NP // 16
_BE = 20000
_GRID = _E // _BE
_EPS = 1e-5
_SELU_A = 1.6732632423543772
_SELU_S = 1.0507009873554805

_INTERPRET = False


def _selu(t):
    return _SELU_S * jnp.where(t > 0, t, _SELU_A * (jnp.exp(t) - 1.0))


def _mesh():
    return plsc.VectorSubcoreMesh(core_axis_name="c", subcore_axis_name="s")


def _sc_gather(a, b, p, row4, col4):
    @functools.partial(
        pl.kernel,
        out_type=(jax.ShapeDtypeStruct((_E, _D), jnp.float32),
                  jax.ShapeDtypeStruct((_E, _D), jnp.float32)),
        mesh=_mesh(),
        scratch_types=[
            pltpu.VMEM((_SLOTS, _GC), jnp.int32),
            pltpu.VMEM((_SLOTS, _GC), jnp.int32),
            pltpu.VMEM((_SLOTS, _GC, _D), jnp.float32),
            pltpu.VMEM((_SLOTS, _GC, _D), jnp.float32),
            pltpu.SemaphoreType.DMA((_SLOTS,)),
            pltpu.SemaphoreType.DMA((_SLOTS,)),
        ],
    )
    def k(a_h, b_h, p_h, row_h, col_h, g_h, g2_h, ridx, cidx, bg, bp,
          sga, sgp):
        wid = lax.axis_index("s") * 2 + lax.axis_index("c")
        base = wid * _EPW

        def body(m, carry):
            pltpu.sync_copy(row_h.at[wid, m], ridx)
            pltpu.sync_copy(col_h.at[wid, m], cidx)
            da = [pltpu.async_copy(a_h.at[ridx.at[j]], bg.at[j],
                                   sga.at[j]) for j in range(_SLOTS)]
            dp = [pltpu.async_copy(p_h.at[ridx.at[j]], bp.at[j],
                                   sgp.at[j]) for j in range(_SLOTS)]
            db = []
            for j in range(_SLOTS):
                da[j].wait()
                db.append(pltpu.async_copy(b_h.at[cidx.at[j]],
                                           bg.at[j], sga.at[j], add=True))
            dw = []
            for j in range(_SLOTS):
                off = base + (m * _SLOTS + j) * _GC
                dp[j].wait()
                dw.append(pltpu.async_copy(bp.at[j],
                                           g2_h.at[pl.ds(off, _GC)],
                                           sgp.at[j]))
                db[j].wait()
                dw.append(pltpu.async_copy(bg.at[j],
                                           g_h.at[pl.ds(off, _GC)],
                                           sga.at[j]))
            for d in dw:
                d.wait()
            return carry

        lax.fori_loop(0, _NCH // _SLOTS, body, 0)

    return k(a, b, p, row4, col4)


def _sc_scatter(h2, col4s, znd):
    @functools.partial(
        pl.kernel,
        out_type=jax.ShapeDtypeStruct((2, _NP, _D), jnp.float32),
        mesh=_mesh(),
        scratch_types=[
            pltpu.VMEM((_SLOTS, _SC_), jnp.int32),
            pltpu.VMEM((_SLOTS, _SC_, _D), jnp.float32),
            pltpu.VMEM_SHARED((_NP, _D), jnp.float32),
            pltpu.SemaphoreType.DMA((_SLOTS,)),
            pltpu.SemaphoreType.DMA((_SLOTS,)),
        ],
    )
    def k(h_h, col_h, z_h, out_h, cidx, vbuf, shared, sld, sad):
        cid = lax.axis_index("c")
        sid = lax.axis_index("s")
        wid = sid * 2 + cid
        pltpu.sync_copy(z_h.at[pl.ds(sid * _NPS, _NPS)],
                        shared.at[pl.ds(sid * _NPS, _NPS)])
        plsc.subcore_barrier()

        def body(m, carry):
            pltpu.sync_copy(col_h.at[wid, m], cidx)
            dl = [pltpu.async_copy(
                h_h.at[pl.ds(wid * _EPW + (m * _SLOTS + j) * _SC_, _SC_)],
                vbuf.at[j], sld.at[j]) for j in range(_SLOTS)]
            da = []
            for j in range(_SLOTS):
                dl[j].wait()
                da.append(pltpu.async_copy(vbuf.at[j],
                                           shared.at[cidx.at[j]],
                                           sad.at[j], add=True))
            for d in da:
                d.wait()
            return carry

        lax.fori_loop(0, _SNCH // _SLOTS, body, 0)
        plsc.subcore_barrier()
        pltpu.sync_copy(shared.at[pl.ds(sid * _NPS, _NPS)],
                        out_h.at[cid, pl.ds(sid * _NPS, _NPS)])

    return k(h2, col4s, znd)


def _sc_counts(col4s, z16, ones16):
    @functools.partial(
        pl.kernel,
        out_type=jax.ShapeDtypeStruct((2, _NP, _D), jnp.float32),
        mesh=_mesh(),
        scratch_types=[
            pltpu.VMEM((_SLOTS, _SC_), jnp.int32),
            pltpu.VMEM((_SC_, _D), jnp.float32),
            pltpu.VMEM_SHARED((_NP, _D), jnp.float32),
            pltpu.SemaphoreType.DMA((_SLOTS,)),
        ],
    )
    def k(col_h, z_h, ones_h, out_h, cidx, obuf, shared, sad):
        cid = lax.axis_index("c")
        sid = lax.axis_index("s")
        wid = sid * 2 + cid
        pltpu.sync_copy(ones_h, obuf)
        pltpu.sync_copy(z_h.at[pl.ds(sid * _NPS, _NPS)],
                        shared.at[pl.ds(sid * _NPS, _NPS)])
        plsc.subcore_barrier()

        def body(m, carry):
            pltpu.sync_copy(col_h.at[wid, m], cidx)
            da = [pltpu.async_copy(obuf, shared.at[cidx.at[j]],
                                   sad.at[j], add=True)
                  for j in range(_SLOTS)]
            for d in da:
                d.wait()
            return carry

        lax.fori_loop(0, _SNCH // _SLOTS, body, 0)
        plsc.subcore_barrier()
        pltpu.sync_copy(shared.at[pl.ds(sid * _NPS, _NPS)],
                        out_h.at[cid, pl.ds(sid * _NPS, _NPS)])

    return k(col4s, z16, ones16)


def _bspec_e(w):
    return pl.BlockSpec((_BE, w), lambda i: (i, 0))


def _bspec_c(shape):
    return pl.BlockSpec(shape, lambda i: (0,) * len(shape))


def _f32(ref):
    return ref[...].astype(jnp.float32)


def _acc_stats(st_ref, z):
    st = jnp.concatenate([jnp.sum(z, axis=0, keepdims=True),
                          jnp.sum(z * z, axis=0, keepdims=True),
                          jnp.zeros((6, _D), jnp.float32)], axis=0)
    i = pl.program_id(0)

    @pl.when(i == 0)
    def _():
        st_ref[...] = st

    @pl.when(i != 0)
    def _():
        st_ref[...] = st_ref[...] + st


def _affine(st_ref, gm_ref, bt_ref):
    m = st_ref[0:1, :] * (1.0 / _E)
    v = st_ref[1:2, :] * (1.0 / _E) - m * m
    sc = gm_ref[...] * lax.rsqrt(v + _EPS)
    return sc, bt_ref[...] - m * sc


def _tc1(g, cin, wc, cv):
    de = cin.shape[1]

    def body(g_ref, c_ref, w_ref, v_ref, z_ref, st_ref):
        z = (_f32(g_ref)
             + jnp.dot(_f32(c_ref), w_ref[...],
                       preferred_element_type=jnp.float32)
             + v_ref[...])
        z_ref[...] = z.astype(jnp.bfloat16)
        _acc_stats(st_ref, z)

    return pl.pallas_call(
        body,
        grid=(_GRID,),
        in_specs=[_bspec_e(_D), _bspec_e(de), _bspec_c((de, _D)),
                  _bspec_c((1, _D))],
        out_specs=[_bspec_e(_D), _bspec_c((8, _D))],
        out_shape=[jax.ShapeDtypeStruct((_E, _D), jnp.bfloat16),
                   jax.ShapeDtypeStruct((8, _D), jnp.float32)],
        interpret=_INTERPRET,
    )(g, cin, wc, cv)


def _tc2(z1, st1, gm, bt, w, b):
    def body(z_ref, st_ref, gm_ref, bt_ref, w_ref, b_ref, o_ref, so_ref):
        sc, sh = _affine(st_ref, gm_ref, bt_ref)
        e1 = _selu(_f32(z_ref) * sc + sh)
        z2 = jnp.dot(e1, w_ref[...],
                     preferred_element_type=jnp.float32) + b_ref[...]
        o_ref[...] = z2.astype(jnp.bfloat16)
        _acc_stats(so_ref, z2)

    return pl.pallas_call(
        body,
        grid=(_GRID,),
        in_specs=[_bspec_e(_D), _bspec_c((8, _D)), _bspec_c((1, _D)),
                  _bspec_c((1, _D)), _bspec_c((_D, _D)), _bspec_c((1, _D))],
        out_specs=[_bspec_e(_D), _bspec_c((8, _D))],
        out_shape=[jax.ShapeDtypeStruct((_E, _D), jnp.bfloat16),
                   jax.ShapeDtypeStruct((8, _D), jnp.float32)],
        interpret=_INTERPRET,
    )(z1, st1, gm, bt, w, b)


def _tc3(z2, g2, st2, gm, bt, w3n, cv2, write_e2):
    def body_full(z_ref, g2_ref, st_ref, gm_ref, bt_ref, w_ref, v_ref,
                  e2_ref, h1_ref, so_ref):
        sc, sh = _affine(st_ref, gm_ref, bt_ref)
        e2 = _selu(_f32(z_ref) * sc + sh)
        e2_ref[...] = e2.astype(jnp.bfloat16)
        h1 = (_f32(g2_ref)
              + jnp.dot(e2, w_ref[...], preferred_element_type=jnp.float32)
              + v_ref[...])
        h1_ref[...] = h1.astype(jnp.bfloat16)
        _acc_stats(so_ref, h1)

    def body_nocarry(z_ref, g2_ref, st_ref, gm_ref, bt_ref, w_ref, v_ref,
                     h1_ref, so_ref):
        sc, sh = _affine(st_ref, gm_ref, bt_ref)
        e2 = _selu(_f32(z_ref) * sc + sh)
        h1 = (_f32(g2_ref)
              + jnp.dot(e2, w_ref[...], preferred_element_type=jnp.float32)
              + v_ref[...])
        h1_ref[...] = h1.astype(jnp.bfloat16)
        _acc_stats(so_ref, h1)

    in_specs = [_bspec_e(_D), _bspec_e(_D), _bspec_c((8, _D)),
                _bspec_c((1, _D)), _bspec_c((1, _D)), _bspec_c((_D, _D)),
                _bspec_c((1, _D))]
    if write_e2:
        return pl.pallas_call(
            body_full,
            grid=(_GRID,),
            in_specs=in_specs,
            out_specs=[_bspec_e(_D), _bspec_e(_D), _bspec_c((8, _D))],
            out_shape=[jax.ShapeDtypeStruct((_E, _D), jnp.bfloat16),
                       jax.ShapeDtypeStruct((_E, _D), jnp.bfloat16),
                       jax.ShapeDtypeStruct((8, _D), jnp.float32)],
            interpret=_INTERPRET,
        )(z2, g2, st2, gm, bt, w3n, cv2)
    h1, st3 = pl.pallas_call(
        body_nocarry,
        grid=(_GRID,),
        in_specs=in_specs,
        out_specs=[_bspec_e(_D), _bspec_c((8, _D))],
        out_shape=[jax.ShapeDtypeStruct((_E, _D), jnp.bfloat16),
                   jax.ShapeDtypeStruct((8, _D), jnp.float32)],
        interpret=_INTERPRET,
    )(z2, g2, st2, gm, bt, w3n, cv2)
    return None, h1, st3


def _tc4(h1, st3, gm, bt, w, b):
    def body(h_ref, st_ref, gm_ref, bt_ref, w_ref, b_ref, o_ref, so_ref):
        sc, sh = _affine(st_ref, gm_ref, bt_ref)
        n1 = _selu(_f32(h_ref) * sc + sh)
        z4 = jnp.dot(n1, w_ref[...],
                     preferred_element_type=jnp.float32) + b_ref[...]
        o_ref[...] = z4.astype(jnp.bfloat16)
        _acc_stats(so_ref, z4)

    return pl.pallas_call(
        body,
        grid=(_GRID,),
        in_specs=[_bspec_e(_D), _bspec_c((8, _D)), _bspec_c((1, _D)),
                  _bspec_c((1, _D)), _bspec_c((_D, _D)), _bspec_c((1, _D))],
        out_specs=[_bspec_e(_D), _bspec_c((8, _D))],
        out_shape=[jax.ShapeDtypeStruct((_E, _D), jnp.bfloat16),
                   jax.ShapeDtypeStruct((8, _D), jnp.float32)],
        interpret=_INTERPRET,
    )(h1, st3, gm, bt, w, b)


def _tc5(z4, st4, gm, bt):
    def body(z_ref, st_ref, gm_ref, bt_ref, o_ref):
        sc, sh = _affine(st_ref, gm_ref, bt_ref)
        o_ref[...] = _selu(_f32(z_ref) * sc + sh)

    return pl.pallas_call(
        body,
        grid=(_GRID,),
        in_specs=[_bspec_e(_D), _bspec_c((8, _D)), _bspec_c((1, _D)),
                  _bspec_c((1, _D))],
        out_specs=[_bspec_e(_D)],
        out_shape=[jax.ShapeDtypeStruct((_E, _D), jnp.float32)],
        interpret=_INTERPRET,
    )(z4, st4, gm, bt)[0]


def _onehot(b_ref):
    return (b_ref[...] == lax.broadcasted_iota(
        jnp.int32, (_N, _NG), 1)).astype(jnp.float32)


def _prep(x, u, b2d, wsrc, wdst, wu, b1v, wnx, bnv):
    def body(x_ref, u_ref, b_ref, ws, wd, wu_, b1_, wn_, bn_,
             a_ref, bb_ref, p_ref):
        xx = x_ref[...]
        oh = _onehot(b_ref)
        up = jnp.dot(u_ref[...], wu_[...], preferred_element_type=jnp.float32)
        a_ref[...] = (jnp.dot(xx, ws[...], preferred_element_type=jnp.float32)
                      + jnp.dot(oh, up, preferred_element_type=jnp.float32)
                      + b1_[...])
        bb_ref[...] = jnp.dot(xx, wd[...], preferred_element_type=jnp.float32)
        p_ref[...] = jnp.dot(xx, wn_[...],
                             preferred_element_type=jnp.float32) + bn_[...]

    return pl.pallas_call(
        body,
        out_shape=[jax.ShapeDtypeStruct((_N, _D), jnp.float32)] * 3,
        interpret=_INTERPRET,
    )(x, u, b2d, wsrc, wdst, wu, b1v, wnx, bnv)


def _bn_rows(t, gm, bt):
    m = jnp.mean(t, axis=0, keepdims=True)
    v = jnp.mean((t - m) ** 2, axis=0, keepdims=True)
    return (t - m) * lax.rsqrt(v + _EPS) * gm + bt


def _node(x, u, b2d, s2p, cp, wn3, bn3v, n2w, gw, nextw):
    has_next = nextw is not None

    def body(*refs):
        (x_ref, u_ref, b_ref, s_ref, c_ref, wn3_, bn3_,
         wax, wag, wau, ba1, ga1, bea1, wa2, ba2, ga2, bea2, wa3, ba3,
         wg1u, wg1n, bg1, gg1, beg1, wg2, bg2, gg2, beg2, wg3, bg3) = refs[:30]
        if has_next:
            nws, nwd, nwu, nb1, nwn, nbn = refs[30:36]
            un_ref, xn_ref, a_ref, bb_ref, p_ref = refs[36:]
        else:
            un_ref = refs[30]

        def dot(a, b):
            return jnp.dot(a, b, preferred_element_type=jnp.float32)

        s2 = s_ref[0, 0:_N, :] + s_ref[1, 0:_N, :]
        cc = c_ref[0, 0:_N, 0:1] + c_ref[1, 0:_N, 0:1]
        agg = (dot(s2, wn3_[...]) + cc * bn3_[...]) / jnp.maximum(cc, 1.0)
        oh = _onehot(b_ref)
        xx = x_ref[...]
        uu = u_ref[...]
        t = (dot(xx, wax[...]) + dot(agg, wag[...])
             + dot(oh, dot(uu, wau[...])) + ba1[...])
        t = _selu(_bn_rows(t, ga1[...], bea1[...]))
        t = dot(t, wa2[...]) + ba2[...]
        t = _selu(_bn_rows(t, ga2[...], bea2[...]))
        xn = dot(t, wa3[...]) + ba3[...]

        nc = lax.dot_general(oh, jnp.full((_N, 1), 1.0, jnp.float32),
                             (((0,), (0,)), ((), ())),
                             preferred_element_type=jnp.float32)
        ns = lax.dot_general(oh, xn, (((0,), (0,)), ((), ())),
                             preferred_element_type=jnp.float32)
        nmean = ns / jnp.maximum(nc, 1.0)
        g = dot(uu, wg1u[...]) + dot(nmean, wg1n[...]) + bg1[...]
        g = _selu(_bn_rows(g, gg1[...], beg1[...]))
        g = dot(g, wg2[...]) + bg2[...]
        g = _selu(_bn_rows(g, gg2[...], beg2[...]))
        un = dot(g, wg3[...]) + bg3[...]
        un_ref[...] = un
        if has_next:
            xn_ref[...] = xn
            a_ref[...] = (dot(xn, nws[...]) + dot(oh, dot(un, nwu[...]))
                          + nb1[...])
            bb_ref[...] = dot(xn, nwd[...])
            p_ref[...] = dot(xn, nwn[...]) + nbn[...]

    out_shape = [jax.ShapeDtypeStruct((_NG, _D), jnp.float32)]
    if has_next:
        out_shape += [jax.ShapeDtypeStruct((_N, _D), jnp.float32)] * 4
        args = (x, u, b2d, s2p, cp, wn3, bn3v) + n2w + gw + nextw
    else:
        args = (x, u, b2d, s2p, cp, wn3, bn3v) + n2w + gw
    res = pl.pallas_call(
        body, out_shape=out_shape,
        compiler_params=pltpu.CompilerParams(
            vmem_limit_bytes=100 * 1024 * 1024),
        interpret=_INTERPRET)(*args)
    return res if has_next else res[0]


def kernel(x, edge_attr, u, params, edge_index, batch):
    f32 = jnp.float32
    row4 = edge_index[0].reshape(_NW, _NCH // _SLOTS, _SLOTS, _GC)
    col4 = edge_index[1].reshape(_NW, _NCH // _SLOTS, _SLOTS, _GC)
    col4s = edge_index[1].reshape(_NW, _SNCH // _SLOTS, _SLOTS, _SC_)
    b2d = batch.reshape(_N, 1)
    znd = jnp.zeros((_NP, _D), f32)
    onesd = jnp.ones((_SC_, _D), f32)
    cp = _sc_counts(col4s, znd, onesd)

    def vec(v):
        return v.reshape(1, -1)

    xl, ul = x, u
    carry = None
    w3p = b3p = None
    nxt = None
    for l in range(3):
        lp = params[l]
        (w1, b1, g1, be1), (w2, b2, g2, be2), (w3, b3) = lp['edge']
        (wn1, bn1, gn1, ben1), (wn2, bn2, gn2, ben2), (wn3, bn3) = lp['node1']
        de = 16 if l == 0 else _D
        w1s, w1d = w1[0:_D], w1[_D:2 * _D]
        w1e, w1u = w1[2 * _D:2 * _D + de], w1[2 * _D + de:]
        if l == 0:
            a, bt, p = _prep(xl, ul, b2d, w1s, w1d, w1u, vec(b1),
                             wn1[:_D], vec(bn1))
            cin, wc, cv1 = edge_attr, w1e, jnp.zeros((1, _D), f32)
        else:
            a, bt, p = nxt
            cin = carry
            wc = w3p @ w1e
            cv1 = vec(b3p @ w1e)
        gth, gth2 = _sc_gather(a, bt, p, row4, col4)
        z1, st1 = _tc1(gth, cin, wc, cv1)
        z2, st2 = _tc2(z1, st1, vec(g1), vec(be1), w2, vec(b2))
        w3n = w3 @ wn1[_D:]
        cv2 = vec(b3 @ wn1[_D:])
        e2, h1, st3 = _tc3(z2, gth2, st2, vec(g2), vec(be2), w3n, cv2, l < 2)
        z4, st4 = _tc4(h1, st3, vec(gn1), vec(ben1), wn2, vec(bn2))
        h2 = _tc5(z4, st4, vec(gn2), vec(ben2))
        s2p = _sc_scatter(h2, col4s, znd)
        (wa1, ba1, ga1, bea1), (wa2, ba2, ga2, bea2), (wa3, ba3) = lp['node2']
        (wg1, bg1, gg1, beg1), (wg2, bg2, gg2, beg2), (wg3, bg3) = lp['global']
        n2w = (wa1[:_D], wa1[_D:2 * _D], wa1[2 * _D:], vec(ba1), vec(ga1),
               vec(bea1), wa2, vec(ba2), vec(ga2), vec(bea2), wa3, vec(ba3))
        gw = (wg1[:_D], wg1[_D:], vec(bg1), vec(gg1), vec(beg1),
              wg2, vec(bg2), vec(gg2), vec(beg2), wg3, vec(bg3))
        if l < 2:
            nlp = params[l + 1]
            nw1, nb1 = nlp['edge'][0][0], nlp['edge'][0][1]
            nwn1, nbn1 = nlp['node1'][0][0], nlp['node1'][0][1]
            nextw = (nw1[0:_D], nw1[_D:2 * _D], nw1[3 * _D:], vec(nb1),
                     nwn1[:_D], vec(nbn1))
            un, xn, na, nb_, np_ = _node(xl, ul, b2d, s2p, cp, wn3, vec(bn3),
                                         n2w, gw, nextw)
            nxt = (na, nb_, np_)
            xl, ul = xn, un
            carry = e2
            w3p, b3p = w3, b3
        else:
            return _node(xl, ul, b2d, s2p, cp, wn3, vec(bn3), n2w, gw, None)
